# Initial kernel scaffold; baseline (speedup 1.0000x reference)
import jax
import jax.numpy as jnp
from jax.experimental import pallas as pl


def kernel(x, edge_index, W_in, b_in, W_c1, b_c1, g_ln1, b_ln1, W_c2, b_c2, g_ln2, b_ln2, W_out, b_out):
    raise NotImplementedError("write your pallas kernel here")


# speedup vs baseline: 9.5799x; 9.5799x over previous
import functools

import jax
import jax.numpy as jnp
from jax import lax
from jax.experimental import pallas as pl
from jax.experimental.pallas import tpu as pltpu
from jax.experimental.pallas import tpu_sc as plsc

N = 10000
E = 160000
D = 256
DH = 128
EPS = 1e-5

NC = 2
NS = 16
RQ = 624
EPSC = E // NS
CH = 128
NFULL = EPSC // CH
TAIL = EPSC - NFULL * CH
EPW = E // (NC * NS)
NFULL_D = EPW // CH
TAIL_D = EPW - NFULL_D * CH

@functools.cache
def _get_mesh():
    return plsc.VectorSubcoreMesh(
        core_axis_name="c", subcore_axis_name="s", num_cores=NC, num_subcores=NS
    )


@functools.cache
def _deg_pass_built():
    return pl.kernel(
        _deg_body,
        mesh=_get_mesh(),
        out_type=jax.ShapeDtypeStruct((2 * N, DH), jnp.float32),
        scratch_types=[
            pltpu.VMEM((CH,), jnp.int32),
            pltpu.VMEM((TAIL_D,), jnp.int32),
            pltpu.VMEM((CH, DH), jnp.float32),
            pltpu.VMEM((TAIL_D, DH), jnp.float32),
            pltpu.VMEM((16, DH), jnp.float32),
            pltpu.VMEM_SHARED((N, DH), jnp.float32),
        ],
    )


def _deg_body(dst_hbm, out_hbm, dst_v, dst_t, ones_v, ones_t, zbuf, acc):
    c = lax.axis_index("c")
    s = lax.axis_index("s")

    def zfill(r, carry):
        for j in range(DH // 16):
            zbuf[r, pl.ds(j * 16, 16)] = jnp.zeros((16,), jnp.float32)
        return carry

    lax.fori_loop(0, 16, zfill, 0)

    def ofill(r, carry):
        for j in range(DH // 16):
            ones_v[r, pl.ds(j * 16, 16)] = jnp.full((16,), 1.0, jnp.float32)
        return carry

    lax.fori_loop(0, CH, ofill, 0)
    for r in range(TAIL_D):
        for j in range(DH // 16):
            ones_t[r, pl.ds(j * 16, 16)] = jnp.full((16,), 1.0, jnp.float32)

    row0 = pl.multiple_of(s * RQ, 8)
    nsteps = jnp.where(s == NS - 1, (N - (NS - 1) * RQ) // 16, RQ // 16)

    def zstep(t, carry):
        pltpu.sync_copy(zbuf, acc.at[pl.ds(row0 + t * 16, 16)])
        return carry

    lax.fori_loop(0, nsteps, zstep, 0)
    plsc.subcore_barrier()

    base0 = (s * NC + c) * EPW

    def chunk(t, carry):
        base = pl.multiple_of(base0 + t * CH, 8)
        pltpu.sync_copy(dst_hbm.at[pl.ds(base, CH)], dst_v)
        pltpu.sync_copy(ones_v, acc.at[dst_v], add=True)
        return carry

    lax.fori_loop(0, NFULL_D, chunk, 0)
    base = pl.multiple_of(base0 + NFULL_D * CH, 8)
    pltpu.sync_copy(dst_hbm.at[pl.ds(base, TAIL_D)], dst_t)
    pltpu.sync_copy(ones_t, acc.at[dst_t], add=True)
    plsc.subcore_barrier()
    pltpu.sync_copy(acc.at[pl.ds(row0, RQ)], out_hbm.at[pl.ds(c * N + row0, RQ)])

    @pl.when(s == NS - 1)
    def _():
        extra = NS * RQ
        nex = N - extra
        pltpu.sync_copy(acc.at[pl.ds(extra, nex)],
                        out_hbm.at[pl.ds(c * N + extra, nex)])


@functools.cache
def _row_pass_built():
    return pl.kernel(
        _row_body,
        mesh=_get_mesh(),
        out_type=[
            jax.ShapeDtypeStruct((N, DH), jnp.float32),
            jax.ShapeDtypeStruct((N, DH), jnp.float32),
        ],
        scratch_types=[
            pltpu.VMEM((CH,), jnp.int32),
            pltpu.VMEM((CH,), jnp.int32),
            pltpu.VMEM((TAIL,), jnp.int32),
            pltpu.VMEM((TAIL,), jnp.int32),
            pltpu.VMEM((CH, DH), jnp.float32),
            pltpu.VMEM((TAIL, DH), jnp.float32),
            pltpu.VMEM((16, DH), jnp.float32),
            pltpu.VMEM_SHARED((N, DH), jnp.float32),
            pltpu.SemaphoreType.DMA,
        ],
    )


def _row_body(hws_a, hws_b, src_hbm, dst_hbm, out_a, out_b,
              src_v, dst_v, src_t, dst_t, rows_v, rows_t, zbuf, acc, sem):
    c = lax.axis_index("c")
    s = lax.axis_index("s")

    def zfill(r, carry):
        for j in range(DH // 16):
            zbuf[r, pl.ds(j * 16, 16)] = jnp.zeros((16,), jnp.float32)
        return carry

    lax.fori_loop(0, 16, zfill, 0)
    row0 = pl.multiple_of(s * RQ, 8)
    nsteps = jnp.where(s == NS - 1, (N - (NS - 1) * RQ) // 16, RQ // 16)

    def zstep(t, carry):
        pltpu.sync_copy(zbuf, acc.at[pl.ds(row0 + t * 16, 16)])
        return carry

    lax.fori_loop(0, nsteps, zstep, 0)
    plsc.subcore_barrier()

    def run(hws_hbm, out_hbm):
        base0 = s * EPSC

        def chunk(t, carry):
            base = pl.multiple_of(base0 + t * CH, 8)
            pltpu.sync_copy(src_hbm.at[pl.ds(base, CH)], src_v)
            pltpu.sync_copy(dst_hbm.at[pl.ds(base, CH)], dst_v)
            pltpu.async_copy(hws_hbm.at[src_v], rows_v, sem).wait()
            pltpu.sync_copy(rows_v, acc.at[dst_v], add=True)
            return carry

        lax.fori_loop(0, NFULL, chunk, 0)
        base = pl.multiple_of(base0 + NFULL * CH, 8)
        pltpu.sync_copy(src_hbm.at[pl.ds(base, TAIL)], src_t)
        pltpu.sync_copy(dst_hbm.at[pl.ds(base, TAIL)], dst_t)
        pltpu.async_copy(hws_hbm.at[src_t], rows_t, sem).wait()
        pltpu.sync_copy(rows_t, acc.at[dst_t], add=True)
        plsc.subcore_barrier()
        pltpu.sync_copy(acc.at[pl.ds(row0, RQ)], out_hbm.at[pl.ds(row0, RQ)])

        @pl.when(s == NS - 1)
        def _():
            extra = NS * RQ
            nex = N - extra
            pltpu.sync_copy(acc.at[pl.ds(extra, nex)],
                            out_hbm.at[pl.ds(extra, nex)])

    @pl.when(c == 0)
    def _():
        run(hws_a, out_a)

    @pl.when(c == 1)
    def _():
        run(hws_b, out_b)


BN = 1000
GRID = N // BN
_CONTRACT = (((1,), (1,)), ((), ()))


def _tc_pre_body(x_ref, win_ref, bin_ref, w1_ref, degp_ref,
                 h0_ref, dinv_ref, hwsa_ref, hwsb_ref):
    x = x_ref[...]
    h0 = jnp.maximum(
        lax.dot_general(x, win_ref[...], _CONTRACT,
                        preferred_element_type=jnp.float32) + bin_ref[...], 0.0)
    dp = degp_ref[...]
    deg = dp[0][:, :1] + dp[1][:, :1] + 1.0
    dinv = lax.rsqrt(deg)
    hw = lax.dot_general(h0, w1_ref[...], _CONTRACT,
                         preferred_element_type=jnp.float32)
    hws = hw * dinv
    h0_ref[...] = h0
    dinv_ref[...] = jnp.broadcast_to(dinv, (BN, DH))
    hwsa_ref[...] = hws[:, :DH]
    hwsb_ref[...] = hws[:, DH:]


_tc_pre = pl.pallas_call(
    _tc_pre_body,
    grid=(GRID,),
    in_specs=[
        pl.BlockSpec((BN, D), lambda i: (i, 0)),
        pl.BlockSpec((D, D), lambda i: (0, 0)),
        pl.BlockSpec((1, D), lambda i: (0, 0)),
        pl.BlockSpec((D, D), lambda i: (0, 0)),
        pl.BlockSpec((2, BN, DH), lambda i: (0, i, 0)),
    ],
    out_specs=[
        pl.BlockSpec((BN, D), lambda i: (i, 0)),
        pl.BlockSpec((BN, DH), lambda i: (i, 0)),
        pl.BlockSpec((BN, DH), lambda i: (i, 0)),
        pl.BlockSpec((BN, DH), lambda i: (i, 0)),
    ],
    out_shape=[
        jax.ShapeDtypeStruct((N, D), jnp.float32),
        jax.ShapeDtypeStruct((N, DH), jnp.float32),
        jax.ShapeDtypeStruct((N, DH), jnp.float32),
        jax.ShapeDtypeStruct((N, DH), jnp.float32),
    ],
)


def _layer_tail(sa, sb, hwsa, hwsb, dinv, hprev, bc, g, bl):
    conv = jnp.concatenate([sa + hwsa, sb + hwsb], axis=1) * dinv + bc
    t = hprev + jnp.maximum(conv, 0.0)
    mu = jnp.mean(t, axis=1, keepdims=True)
    var = jnp.mean((t - mu) ** 2, axis=1, keepdims=True)
    return (t - mu) * lax.rsqrt(var + EPS) * g + bl


def _tc_mid_body(sa_ref, sb_ref, hwsa_ref, hwsb_ref, dinv_ref, hprev_ref,
                 bc_ref, g_ref, bl_ref, w2_ref,
                 h1_ref, h2a_ref, h2b_ref):
    dinv = dinv_ref[...][:, :1]
    h1 = _layer_tail(sa_ref[...], sb_ref[...], hwsa_ref[...], hwsb_ref[...],
                     dinv, hprev_ref[...], bc_ref[...], g_ref[...], bl_ref[...])
    h1_ref[...] = h1
    hw2 = lax.dot_general(h1, w2_ref[...], _CONTRACT,
                          preferred_element_type=jnp.float32)
    hws2 = hw2 * dinv
    h2a_ref[...] = hws2[:, :DH]
    h2b_ref[...] = hws2[:, DH:]


_tc_mid = pl.pallas_call(
    _tc_mid_body,
    grid=(GRID,),
    in_specs=[
        pl.BlockSpec((BN, DH), lambda i: (i, 0)),
        pl.BlockSpec((BN, DH), lambda i: (i, 0)),
        pl.BlockSpec((BN, DH), lambda i: (i, 0)),
        pl.BlockSpec((BN, DH), lambda i: (i, 0)),
        pl.BlockSpec((BN, DH), lambda i: (i, 0)),
        pl.BlockSpec((BN, D), lambda i: (i, 0)),
        pl.BlockSpec((1, D), lambda i: (0, 0)),
        pl.BlockSpec((1, D), lambda i: (0, 0)),
        pl.BlockSpec((1, D), lambda i: (0, 0)),
        pl.BlockSpec((D, D), lambda i: (0, 0)),
    ],
    out_specs=[
        pl.BlockSpec((BN, D), lambda i: (i, 0)),
        pl.BlockSpec((BN, DH), lambda i: (i, 0)),
        pl.BlockSpec((BN, DH), lambda i: (i, 0)),
    ],
    out_shape=[
        jax.ShapeDtypeStruct((N, D), jnp.float32),
        jax.ShapeDtypeStruct((N, DH), jnp.float32),
        jax.ShapeDtypeStruct((N, DH), jnp.float32),
    ],
)


def _tc_fin_body(sa_ref, sb_ref, hwsa_ref, hwsb_ref, dinv_ref, hprev_ref,
                 bc_ref, g_ref, bl_ref, wo_ref, bo_ref, out_ref):
    dinv = dinv_ref[...][:, :1]
    h2 = _layer_tail(sa_ref[...], sb_ref[...], hwsa_ref[...], hwsb_ref[...],
                     dinv, hprev_ref[...], bc_ref[...], g_ref[...], bl_ref[...])
    out_ref[...] = lax.dot_general(
        h2, wo_ref[...], _CONTRACT, preferred_element_type=jnp.float32
    ) + bo_ref[...]


_tc_fin = pl.pallas_call(
    _tc_fin_body,
    grid=(GRID,),
    in_specs=[
        pl.BlockSpec((BN, DH), lambda i: (i, 0)),
        pl.BlockSpec((BN, DH), lambda i: (i, 0)),
        pl.BlockSpec((BN, DH), lambda i: (i, 0)),
        pl.BlockSpec((BN, DH), lambda i: (i, 0)),
        pl.BlockSpec((BN, DH), lambda i: (i, 0)),
        pl.BlockSpec((BN, D), lambda i: (i, 0)),
        pl.BlockSpec((1, D), lambda i: (0, 0)),
        pl.BlockSpec((1, D), lambda i: (0, 0)),
        pl.BlockSpec((1, D), lambda i: (0, 0)),
        pl.BlockSpec((D, D), lambda i: (0, 0)),
        pl.BlockSpec((1, D), lambda i: (0, 0)),
    ],
    out_specs=pl.BlockSpec((BN, D), lambda i: (i, 0)),
    out_shape=jax.ShapeDtypeStruct((N, D), jnp.float32),
)


def kernel(x, edge_index, W_in, b_in, W_c1, b_c1, g_ln1, b_ln1,
           W_c2, b_c2, g_ln2, b_ln2, W_out, b_out):
    src = edge_index[0]
    dst = edge_index[1]

    degp = _deg_pass_built()(dst).reshape(2, N, DH)
    h0, dinv, hws1a, hws1b = _tc_pre(
        x, W_in, b_in.reshape(1, D), W_c1, degp)
    s1a, s1b = _row_pass_built()(hws1a, hws1b, src, dst)
    h1, hws2a, hws2b = _tc_mid(
        s1a, s1b, hws1a, hws1b, dinv, h0,
        b_c1.reshape(1, D), g_ln1.reshape(1, D), b_ln1.reshape(1, D), W_c2)
    s2a, s2b = _row_pass_built()(hws2a, hws2b, src, dst)
    out = _tc_fin(
        s2a, s2b, hws2a, hws2b, dinv, h1,
        b_c2.reshape(1, D), g_ln2.reshape(1, D), b_ln2.reshape(1, D),
        W_out, b_out.reshape(1, D))
    return out


# speedup vs baseline: 13.3055x; 1.3889x over previous
import functools

import jax
import jax.numpy as jnp
from jax import lax
from jax.experimental import pallas as pl
from jax.experimental.pallas import tpu as pltpu
from jax.experimental.pallas import tpu_sc as plsc

N = 10000
E = 160000
D = 256
DH = 128
EPS = 1e-5

NC = 2
NS = 16
RQ = 624
EPSC = E // NS
CH = 128
NFULL = EPSC // CH
TAIL = EPSC - NFULL * CH
EPW = E // (NC * NS)
NFULL_D = EPW // CH
TAIL_D = EPW - NFULL_D * CH

@functools.cache
def _get_mesh():
    return plsc.VectorSubcoreMesh(
        core_axis_name="c", subcore_axis_name="s", num_cores=NC, num_subcores=NS
    )


@functools.cache
def _deg_pass_built():
    return pl.kernel(
        _deg_body,
        mesh=_get_mesh(),
        out_type=jax.ShapeDtypeStruct((2 * N, DH), jnp.float32),
        scratch_types=[
            pltpu.VMEM((CH,), jnp.int32),
            pltpu.VMEM((TAIL_D,), jnp.int32),
            pltpu.VMEM((CH, DH), jnp.float32),
            pltpu.VMEM((TAIL_D, DH), jnp.float32),
            pltpu.VMEM((16, DH), jnp.float32),
            pltpu.VMEM_SHARED((N, DH), jnp.float32),
        ],
    )


def _deg_body(dst_hbm, out_hbm, dst_v, dst_t, ones_v, ones_t, zbuf, acc):
    c = lax.axis_index("c")
    s = lax.axis_index("s")

    def zfill(r, carry):
        for j in range(DH // 16):
            zbuf[r, pl.ds(j * 16, 16)] = jnp.zeros((16,), jnp.float32)
        return carry

    lax.fori_loop(0, 16, zfill, 0)

    def ofill(r, carry):
        for j in range(DH // 16):
            ones_v[r, pl.ds(j * 16, 16)] = jnp.full((16,), 1.0, jnp.float32)
        return carry

    lax.fori_loop(0, CH, ofill, 0)
    for r in range(TAIL_D):
        for j in range(DH // 16):
            ones_t[r, pl.ds(j * 16, 16)] = jnp.full((16,), 1.0, jnp.float32)

    row0 = pl.multiple_of(s * RQ, 8)
    nsteps = jnp.where(s == NS - 1, (N - (NS - 1) * RQ) // 16, RQ // 16)

    def zstep(t, carry):
        pltpu.sync_copy(zbuf, acc.at[pl.ds(row0 + t * 16, 16)])
        return carry

    lax.fori_loop(0, nsteps, zstep, 0)
    plsc.subcore_barrier()

    base0 = (s * NC + c) * EPW

    def chunk(t, carry):
        base = pl.multiple_of(base0 + t * CH, 8)
        pltpu.sync_copy(dst_hbm.at[pl.ds(base, CH)], dst_v)
        pltpu.sync_copy(ones_v, acc.at[dst_v], add=True)
        return carry

    lax.fori_loop(0, NFULL_D, chunk, 0)
    base = pl.multiple_of(base0 + NFULL_D * CH, 8)
    pltpu.sync_copy(dst_hbm.at[pl.ds(base, TAIL_D)], dst_t)
    pltpu.sync_copy(ones_t, acc.at[dst_t], add=True)
    plsc.subcore_barrier()
    pltpu.sync_copy(acc.at[pl.ds(row0, RQ)], out_hbm.at[pl.ds(c * N + row0, RQ)])

    @pl.when(s == NS - 1)
    def _():
        extra = NS * RQ
        nex = N - extra
        pltpu.sync_copy(acc.at[pl.ds(extra, nex)],
                        out_hbm.at[pl.ds(c * N + extra, nex)])


CHK = 80
NCHS = EPSC // CHK
NPAIR = (NCHS - 1) // 2


@functools.cache
def _row_pass_built():
    return pl.kernel(
        _row_body,
        mesh=_get_mesh(),
        out_type=[
            jax.ShapeDtypeStruct((N, DH), jnp.float32),
            jax.ShapeDtypeStruct((N, DH), jnp.float32),
        ],
        scratch_types=[
            pltpu.VMEM((EPSC,), jnp.int32),
            pltpu.VMEM((CHK,), jnp.int32),
            pltpu.VMEM((CHK,), jnp.int32),
            pltpu.VMEM((CHK, DH), jnp.float32),
            pltpu.VMEM((CHK, DH), jnp.float32),
            pltpu.VMEM((16, DH), jnp.float32),
            pltpu.VMEM_SHARED((N, DH), jnp.float32),
            pltpu.SemaphoreType.DMA,
            pltpu.SemaphoreType.DMA,
            pltpu.SemaphoreType.DMA,
            pltpu.SemaphoreType.DMA,
        ],
    )


def _row_body(hws_a, hws_b, src_hbm, dst_hbm, out_a, out_b,
              src_v, dst_pa, dst_pb, rows_pa, rows_pb, zbuf, acc,
              sem_ga, sem_gb, sem_sa, sem_sb):
    c = lax.axis_index("c")
    s = lax.axis_index("s")

    def zfill(r, carry):
        for j in range(DH // 16):
            zbuf[r, pl.ds(j * 16, 16)] = jnp.zeros((16,), jnp.float32)
        return carry

    lax.fori_loop(0, 16, zfill, 0)
    row0 = pl.multiple_of(s * RQ, 8)
    nsteps = jnp.where(s == NS - 1, (N - (NS - 1) * RQ) // 16, RQ // 16)

    def zstep(t, carry):
        pltpu.sync_copy(zbuf, acc.at[pl.ds(row0 + t * 16, 16)])
        return carry

    lax.fori_loop(0, nsteps, zstep, 0)

    base_e = pl.multiple_of(s * EPSC, 8)
    pltpu.sync_copy(src_hbm.at[pl.ds(base_e, EPSC)], src_v)
    plsc.subcore_barrier()

    def run(hws_hbm, out_hbm):
        def dma_g(t, dstb, rows, semg, issue):
            off = pl.multiple_of(base_e + t * CHK, 8)
            idx = src_v.at[pl.ds(pl.multiple_of(t * CHK, 8), CHK)]
            if issue:
                pltpu.async_copy(dst_hbm.at[pl.ds(off, CHK)], dstb, semg)
                pltpu.async_copy(hws_hbm.at[idx], rows, semg)
            else:
                pltpu.make_async_copy(dst_hbm.at[pl.ds(off, CHK)], dstb, semg).wait()
                pltpu.make_async_copy(hws_hbm.at[idx], rows, semg).wait()

        def dma_s(dstb, rows, sems, issue):
            if issue:
                pltpu.async_copy(rows, acc.at[dstb], sems, add=True)
            else:
                pltpu.make_async_copy(rows, acc.at[dstb], sems).wait()

        dma_g(0, dst_pa, rows_pa, sem_ga, True)
        dma_g(0, dst_pa, rows_pa, sem_ga, False)
        dma_s(dst_pa, rows_pa, sem_sa, True)
        dma_g(1, dst_pb, rows_pb, sem_gb, True)

        def pair(k, carry):
            tb = 2 * k + 1
            ta = 2 * k + 2
            dma_g(tb, dst_pb, rows_pb, sem_gb, False)
            dma_s(dst_pa, rows_pa, sem_sa, False)
            dma_s(dst_pb, rows_pb, sem_sb, True)
            dma_g(ta, dst_pa, rows_pa, sem_ga, True)
            dma_g(ta, dst_pa, rows_pa, sem_ga, False)
            dma_s(dst_pb, rows_pb, sem_sb, False)
            dma_s(dst_pa, rows_pa, sem_sa, True)

            @pl.when(k < NPAIR - 1)
            def _():
                dma_g(ta + 1, dst_pb, rows_pb, sem_gb, True)

            return carry

        lax.fori_loop(0, NPAIR, pair, 0)
        dma_s(dst_pa, rows_pa, sem_sa, False)
        plsc.subcore_barrier()
        pltpu.sync_copy(acc.at[pl.ds(row0, RQ)], out_hbm.at[pl.ds(row0, RQ)])

        @pl.when(s == NS - 1)
        def _():
            extra = NS * RQ
            nex = N - extra
            pltpu.sync_copy(acc.at[pl.ds(extra, nex)],
                            out_hbm.at[pl.ds(extra, nex)])

    @pl.when(c == 0)
    def _():
        run(hws_a, out_a)

    @pl.when(c == 1)
    def _():
        run(hws_b, out_b)


BN = 1000
GRID = N // BN
_CONTRACT = (((1,), (1,)), ((), ()))


def _tc_pre_body(x_ref, win_ref, bin_ref, w1_ref, degp_ref,
                 h0_ref, dinv_ref, hwsa_ref, hwsb_ref):
    x = x_ref[...]
    h0 = jnp.maximum(
        lax.dot_general(x, win_ref[...], _CONTRACT,
                        preferred_element_type=jnp.float32) + bin_ref[...], 0.0)
    dp = degp_ref[...]
    deg = dp[0][:, :1] + dp[1][:, :1] + 1.0
    dinv = lax.rsqrt(deg)
    hw = lax.dot_general(h0, w1_ref[...], _CONTRACT,
                         preferred_element_type=jnp.float32)
    hws = hw * dinv
    h0_ref[...] = h0
    dinv_ref[...] = jnp.broadcast_to(dinv, (BN, DH))
    hwsa_ref[...] = hws[:, :DH]
    hwsb_ref[...] = hws[:, DH:]


_tc_pre = pl.pallas_call(
    _tc_pre_body,
    grid=(GRID,),
    in_specs=[
        pl.BlockSpec((BN, D), lambda i: (i, 0)),
        pl.BlockSpec((D, D), lambda i: (0, 0)),
        pl.BlockSpec((1, D), lambda i: (0, 0)),
        pl.BlockSpec((D, D), lambda i: (0, 0)),
        pl.BlockSpec((2, BN, DH), lambda i: (0, i, 0)),
    ],
    out_specs=[
        pl.BlockSpec((BN, D), lambda i: (i, 0)),
        pl.BlockSpec((BN, DH), lambda i: (i, 0)),
        pl.BlockSpec((BN, DH), lambda i: (i, 0)),
        pl.BlockSpec((BN, DH), lambda i: (i, 0)),
    ],
    out_shape=[
        jax.ShapeDtypeStruct((N, D), jnp.float32),
        jax.ShapeDtypeStruct((N, DH), jnp.float32),
        jax.ShapeDtypeStruct((N, DH), jnp.float32),
        jax.ShapeDtypeStruct((N, DH), jnp.float32),
    ],
)


def _layer_tail(sa, sb, hwsa, hwsb, dinv, hprev, bc, g, bl):
    conv = jnp.concatenate([sa + hwsa, sb + hwsb], axis=1) * dinv + bc
    t = hprev + jnp.maximum(conv, 0.0)
    mu = jnp.mean(t, axis=1, keepdims=True)
    var = jnp.mean((t - mu) ** 2, axis=1, keepdims=True)
    return (t - mu) * lax.rsqrt(var + EPS) * g + bl


def _tc_mid_body(sa_ref, sb_ref, hwsa_ref, hwsb_ref, dinv_ref, hprev_ref,
                 bc_ref, g_ref, bl_ref, w2_ref,
                 h1_ref, h2a_ref, h2b_ref):
    dinv = dinv_ref[...][:, :1]
    h1 = _layer_tail(sa_ref[...], sb_ref[...], hwsa_ref[...], hwsb_ref[...],
                     dinv, hprev_ref[...], bc_ref[...], g_ref[...], bl_ref[...])
    h1_ref[...] = h1
    hw2 = lax.dot_general(h1, w2_ref[...], _CONTRACT,
                          preferred_element_type=jnp.float32)
    hws2 = hw2 * dinv
    h2a_ref[...] = hws2[:, :DH]
    h2b_ref[...] = hws2[:, DH:]


_tc_mid = pl.pallas_call(
    _tc_mid_body,
    grid=(GRID,),
    in_specs=[
        pl.BlockSpec((BN, DH), lambda i: (i, 0)),
        pl.BlockSpec((BN, DH), lambda i: (i, 0)),
        pl.BlockSpec((BN, DH), lambda i: (i, 0)),
        pl.BlockSpec((BN, DH), lambda i: (i, 0)),
        pl.BlockSpec((BN, DH), lambda i: (i, 0)),
        pl.BlockSpec((BN, D), lambda i: (i, 0)),
        pl.BlockSpec((1, D), lambda i: (0, 0)),
        pl.BlockSpec((1, D), lambda i: (0, 0)),
        pl.BlockSpec((1, D), lambda i: (0, 0)),
        pl.BlockSpec((D, D), lambda i: (0, 0)),
    ],
    out_specs=[
        pl.BlockSpec((BN, D), lambda i: (i, 0)),
        pl.BlockSpec((BN, DH), lambda i: (i, 0)),
        pl.BlockSpec((BN, DH), lambda i: (i, 0)),
    ],
    out_shape=[
        jax.ShapeDtypeStruct((N, D), jnp.float32),
        jax.ShapeDtypeStruct((N, DH), jnp.float32),
        jax.ShapeDtypeStruct((N, DH), jnp.float32),
    ],
)


def _tc_fin_body(sa_ref, sb_ref, hwsa_ref, hwsb_ref, dinv_ref, hprev_ref,
                 bc_ref, g_ref, bl_ref, wo_ref, bo_ref, out_ref):
    dinv = dinv_ref[...][:, :1]
    h2 = _layer_tail(sa_ref[...], sb_ref[...], hwsa_ref[...], hwsb_ref[...],
                     dinv, hprev_ref[...], bc_ref[...], g_ref[...], bl_ref[...])
    out_ref[...] = lax.dot_general(
        h2, wo_ref[...], _CONTRACT, preferred_element_type=jnp.float32
    ) + bo_ref[...]


_tc_fin = pl.pallas_call(
    _tc_fin_body,
    grid=(GRID,),
    in_specs=[
        pl.BlockSpec((BN, DH), lambda i: (i, 0)),
        pl.BlockSpec((BN, DH), lambda i: (i, 0)),
        pl.BlockSpec((BN, DH), lambda i: (i, 0)),
        pl.BlockSpec((BN, DH), lambda i: (i, 0)),
        pl.BlockSpec((BN, DH), lambda i: (i, 0)),
        pl.BlockSpec((BN, D), lambda i: (i, 0)),
        pl.BlockSpec((1, D), lambda i: (0, 0)),
        pl.BlockSpec((1, D), lambda i: (0, 0)),
        pl.BlockSpec((1, D), lambda i: (0, 0)),
        pl.BlockSpec((D, D), lambda i: (0, 0)),
        pl.BlockSpec((1, D), lambda i: (0, 0)),
    ],
    out_specs=pl.BlockSpec((BN, D), lambda i: (i, 0)),
    out_shape=jax.ShapeDtypeStruct((N, D), jnp.float32),
)


def kernel(x, edge_index, W_in, b_in, W_c1, b_c1, g_ln1, b_ln1,
           W_c2, b_c2, g_ln2, b_ln2, W_out, b_out):
    src = edge_index[0]
    dst = edge_index[1]

    degp = _deg_pass_built()(dst).reshape(2, N, DH)
    h0, dinv, hws1a, hws1b = _tc_pre(
        x, W_in, b_in.reshape(1, D), W_c1, degp)
    s1a, s1b = _row_pass_built()(hws1a, hws1b, src, dst)
    h1, hws2a, hws2b = _tc_mid(
        s1a, s1b, hws1a, hws1b, dinv, h0,
        b_c1.reshape(1, D), g_ln1.reshape(1, D), b_ln1.reshape(1, D), W_c2)
    s2a, s2b = _row_pass_built()(hws2a, hws2b, src, dst)
    out = _tc_fin(
        s2a, s2b, hws2a, hws2b, dinv, h1,
        b_c2.reshape(1, D), g_ln2.reshape(1, D), b_ln2.reshape(1, D),
        W_out, b_out.reshape(1, D))
    return out


# speedup vs baseline: 17.7699x; 1.3355x over previous
import functools

import jax
import jax.numpy as jnp
from jax import lax
from jax.experimental import pallas as pl
from jax.experimental.pallas import tpu as pltpu
from jax.experimental.pallas import tpu_sc as plsc

N = 10000
E = 160000
D = 256
DH = 128
EPS = 1e-5

NC = 2
NS = 16
RQ = 624
EPSC = E // NS
CH = 128
NFULL = EPSC // CH
TAIL = EPSC - NFULL * CH
EPW = E // (NC * NS)
NFULL_D = EPW // CH
TAIL_D = EPW - NFULL_D * CH

@functools.cache
def _get_mesh():
    return plsc.VectorSubcoreMesh(
        core_axis_name="c", subcore_axis_name="s", num_cores=NC, num_subcores=NS
    )


@functools.cache
def _deg_pass_built():
    return pl.kernel(
        _deg_body,
        mesh=_get_mesh(),
        out_type=jax.ShapeDtypeStruct((2 * N, DH), jnp.float32),
        scratch_types=[
            pltpu.VMEM((CH,), jnp.int32),
            pltpu.VMEM((TAIL_D,), jnp.int32),
            pltpu.VMEM((CH, DH), jnp.float32),
            pltpu.VMEM((TAIL_D, DH), jnp.float32),
            pltpu.VMEM((16, DH), jnp.float32),
            pltpu.VMEM_SHARED((N, DH), jnp.float32),
        ],
    )


def _deg_body(dst_hbm, out_hbm, dst_v, dst_t, ones_v, ones_t, zbuf, acc):
    c = lax.axis_index("c")
    s = lax.axis_index("s")

    def zfill(r, carry):
        for j in range(DH // 16):
            zbuf[r, pl.ds(j * 16, 16)] = jnp.zeros((16,), jnp.float32)
        return carry

    lax.fori_loop(0, 16, zfill, 0)

    def ofill(r, carry):
        for j in range(DH // 16):
            ones_v[r, pl.ds(j * 16, 16)] = jnp.full((16,), 1.0, jnp.float32)
        return carry

    lax.fori_loop(0, CH, ofill, 0)
    for r in range(TAIL_D):
        for j in range(DH // 16):
            ones_t[r, pl.ds(j * 16, 16)] = jnp.full((16,), 1.0, jnp.float32)

    row0 = pl.multiple_of(s * RQ, 8)
    nsteps = jnp.where(s == NS - 1, (N - (NS - 1) * RQ) // 16, RQ // 16)

    def zstep(t, carry):
        pltpu.sync_copy(zbuf, acc.at[pl.ds(row0 + t * 16, 16)])
        return carry

    lax.fori_loop(0, nsteps, zstep, 0)
    plsc.subcore_barrier()

    base0 = (s * NC + c) * EPW

    def chunk(t, carry):
        base = pl.multiple_of(base0 + t * CH, 8)
        pltpu.sync_copy(dst_hbm.at[pl.ds(base, CH)], dst_v)
        pltpu.sync_copy(ones_v, acc.at[dst_v], add=True)
        return carry

    lax.fori_loop(0, NFULL_D, chunk, 0)
    base = pl.multiple_of(base0 + NFULL_D * CH, 8)
    pltpu.sync_copy(dst_hbm.at[pl.ds(base, TAIL_D)], dst_t)
    pltpu.sync_copy(ones_t, acc.at[dst_t], add=True)
    plsc.subcore_barrier()
    pltpu.sync_copy(acc.at[pl.ds(row0, RQ)], out_hbm.at[pl.ds(c * N + row0, RQ)])

    @pl.when(s == NS - 1)
    def _():
        extra = NS * RQ
        nex = N - extra
        pltpu.sync_copy(acc.at[pl.ds(extra, nex)],
                        out_hbm.at[pl.ds(c * N + extra, nex)])


CHK = 80
NCHS = EPSC // CHK
NTRI = (NCHS - 2) // 3


@functools.cache
def _row_pass_built():
    return pl.kernel(
        _row_body,
        mesh=_get_mesh(),
        out_type=[
            jax.ShapeDtypeStruct((N, DH), jnp.float32),
            jax.ShapeDtypeStruct((N, DH), jnp.float32),
        ],
        scratch_types=[
            pltpu.VMEM((EPSC,), jnp.int32),
            pltpu.VMEM((CHK,), jnp.int32),
            pltpu.VMEM((CHK,), jnp.int32),
            pltpu.VMEM((CHK,), jnp.int32),
            pltpu.VMEM((CHK, DH), jnp.float32),
            pltpu.VMEM((CHK, DH), jnp.float32),
            pltpu.VMEM((CHK, DH), jnp.float32),
            pltpu.VMEM((16, DH), jnp.float32),
            pltpu.VMEM_SHARED((N, DH), jnp.float32),
            pltpu.SemaphoreType.DMA,
            pltpu.SemaphoreType.DMA,
            pltpu.SemaphoreType.DMA,
            pltpu.SemaphoreType.DMA,
            pltpu.SemaphoreType.DMA,
            pltpu.SemaphoreType.DMA,
        ],
    )


def _row_body(hws_a, hws_b, src_hbm, dst_hbm, out_a, out_b,
              src_v, dst_p0, dst_p1, dst_p2, rows_p0, rows_p1, rows_p2,
              zbuf, acc,
              sem_g0, sem_g1, sem_g2, sem_s0, sem_s1, sem_s2):
    c = lax.axis_index("c")
    s = lax.axis_index("s")

    def zfill(r, carry):
        for j in range(DH // 16):
            zbuf[r, pl.ds(j * 16, 16)] = jnp.zeros((16,), jnp.float32)
        return carry

    lax.fori_loop(0, 16, zfill, 0)
    row0 = pl.multiple_of(s * RQ, 8)
    nsteps = jnp.where(s == NS - 1, (N - (NS - 1) * RQ) // 16, RQ // 16)

    def zstep(t, carry):
        pltpu.sync_copy(zbuf, acc.at[pl.ds(row0 + t * 16, 16)])
        return carry

    lax.fori_loop(0, nsteps, zstep, 0)

    base_e = pl.multiple_of(s * EPSC, 8)
    pltpu.sync_copy(src_hbm.at[pl.ds(base_e, EPSC)], src_v)
    plsc.subcore_barrier()

    def run(hws_hbm, out_hbm):
        def dma_g(t, dstb, rows, semg, issue):
            off = pl.multiple_of(base_e + t * CHK, 8)
            idx = src_v.at[pl.ds(pl.multiple_of(t * CHK, 8), CHK)]
            if issue:
                pltpu.async_copy(dst_hbm.at[pl.ds(off, CHK)], dstb, semg)
                pltpu.async_copy(hws_hbm.at[idx], rows, semg)
            else:
                pltpu.make_async_copy(dst_hbm.at[pl.ds(off, CHK)], dstb, semg).wait()
                pltpu.make_async_copy(hws_hbm.at[idx], rows, semg).wait()

        def dma_s(dstb, rows, sems, issue):
            if issue:
                pltpu.async_copy(rows, acc.at[dstb], sems, add=True)
            else:
                pltpu.make_async_copy(rows, acc.at[dstb], sems).wait()

        bufs = ((dst_p0, rows_p0, sem_g0, sem_s0),
                (dst_p1, rows_p1, sem_g1, sem_s1),
                (dst_p2, rows_p2, sem_g2, sem_s2))

        def step(t, b, bprev):
            dma_g(t, b[0], b[1], b[2], False)
            dma_s(bprev[0], bprev[1], bprev[3], False)
            dma_s(b[0], b[1], b[3], True)

            @pl.when(t + 2 < NCHS)
            def _():
                dma_g(t + 2, bprev[0], bprev[1], bprev[2], True)

        dma_g(0, dst_p0, rows_p0, sem_g0, True)
        dma_g(1, dst_p1, rows_p1, sem_g1, True)
        dma_g(0, dst_p0, rows_p0, sem_g0, False)
        dma_s(dst_p0, rows_p0, sem_s0, True)
        dma_g(2, dst_p2, rows_p2, sem_g2, True)
        step(1, bufs[1], bufs[0])

        def tri(k, carry):
            t = 3 * k + 2
            step(t, bufs[2], bufs[1])
            step(t + 1, bufs[0], bufs[2])
            step(t + 2, bufs[1], bufs[0])
            return carry

        lax.fori_loop(0, NTRI, tri, 0)
        dma_s(dst_p1, rows_p1, sem_s1, False)
        plsc.subcore_barrier()
        pltpu.sync_copy(acc.at[pl.ds(row0, RQ)], out_hbm.at[pl.ds(row0, RQ)])

        @pl.when(s == NS - 1)
        def _():
            extra = NS * RQ
            nex = N - extra
            pltpu.sync_copy(acc.at[pl.ds(extra, nex)],
                            out_hbm.at[pl.ds(extra, nex)])

    @pl.when(c == 0)
    def _():
        run(hws_a, out_a)

    @pl.when(c == 1)
    def _():
        run(hws_b, out_b)


BN = 1000
GRID = N // BN
_CONTRACT = (((1,), (1,)), ((), ()))


def _tc_pre_body(x_ref, win_ref, bin_ref, w1_ref, degp_ref,
                 h0_ref, dinv_ref, hwsa_ref, hwsb_ref):
    x = x_ref[...]
    h0 = jnp.maximum(
        lax.dot_general(x, win_ref[...], _CONTRACT,
                        preferred_element_type=jnp.float32) + bin_ref[...], 0.0)
    dp = degp_ref[...]
    deg = dp[0][:, :1] + dp[1][:, :1] + 1.0
    dinv = lax.rsqrt(deg)
    hw = lax.dot_general(h0, w1_ref[...], _CONTRACT,
                         preferred_element_type=jnp.float32)
    hws = hw * dinv
    h0_ref[...] = h0
    dinv_ref[...] = jnp.broadcast_to(dinv, (BN, DH))
    hwsa_ref[...] = hws[:, :DH]
    hwsb_ref[...] = hws[:, DH:]


_tc_pre = pl.pallas_call(
    _tc_pre_body,
    grid=(GRID,),
    in_specs=[
        pl.BlockSpec((BN, D), lambda i: (i, 0)),
        pl.BlockSpec((D, D), lambda i: (0, 0)),
        pl.BlockSpec((1, D), lambda i: (0, 0)),
        pl.BlockSpec((D, D), lambda i: (0, 0)),
        pl.BlockSpec((2, BN, DH), lambda i: (0, i, 0)),
    ],
    out_specs=[
        pl.BlockSpec((BN, D), lambda i: (i, 0)),
        pl.BlockSpec((BN, DH), lambda i: (i, 0)),
        pl.BlockSpec((BN, DH), lambda i: (i, 0)),
        pl.BlockSpec((BN, DH), lambda i: (i, 0)),
    ],
    out_shape=[
        jax.ShapeDtypeStruct((N, D), jnp.float32),
        jax.ShapeDtypeStruct((N, DH), jnp.float32),
        jax.ShapeDtypeStruct((N, DH), jnp.float32),
        jax.ShapeDtypeStruct((N, DH), jnp.float32),
    ],
)


def _layer_tail(sa, sb, hwsa, hwsb, dinv, hprev, bc, g, bl):
    conv = jnp.concatenate([sa + hwsa, sb + hwsb], axis=1) * dinv + bc
    t = hprev + jnp.maximum(conv, 0.0)
    mu = jnp.mean(t, axis=1, keepdims=True)
    var = jnp.mean((t - mu) ** 2, axis=1, keepdims=True)
    return (t - mu) * lax.rsqrt(var + EPS) * g + bl


def _tc_mid_body(sa_ref, sb_ref, hwsa_ref, hwsb_ref, dinv_ref, hprev_ref,
                 bc_ref, g_ref, bl_ref, w2_ref,
                 h1_ref, h2a_ref, h2b_ref):
    dinv = dinv_ref[...][:, :1]
    h1 = _layer_tail(sa_ref[...], sb_ref[...], hwsa_ref[...], hwsb_ref[...],
                     dinv, hprev_ref[...], bc_ref[...], g_ref[...], bl_ref[...])
    h1_ref[...] = h1
    hw2 = lax.dot_general(h1, w2_ref[...], _CONTRACT,
                          preferred_element_type=jnp.float32)
    hws2 = hw2 * dinv
    h2a_ref[...] = hws2[:, :DH]
    h2b_ref[...] = hws2[:, DH:]


_tc_mid = pl.pallas_call(
    _tc_mid_body,
    grid=(GRID,),
    in_specs=[
        pl.BlockSpec((BN, DH), lambda i: (i, 0)),
        pl.BlockSpec((BN, DH), lambda i: (i, 0)),
        pl.BlockSpec((BN, DH), lambda i: (i, 0)),
        pl.BlockSpec((BN, DH), lambda i: (i, 0)),
        pl.BlockSpec((BN, DH), lambda i: (i, 0)),
        pl.BlockSpec((BN, D), lambda i: (i, 0)),
        pl.BlockSpec((1, D), lambda i: (0, 0)),
        pl.BlockSpec((1, D), lambda i: (0, 0)),
        pl.BlockSpec((1, D), lambda i: (0, 0)),
        pl.BlockSpec((D, D), lambda i: (0, 0)),
    ],
    out_specs=[
        pl.BlockSpec((BN, D), lambda i: (i, 0)),
        pl.BlockSpec((BN, DH), lambda i: (i, 0)),
        pl.BlockSpec((BN, DH), lambda i: (i, 0)),
    ],
    out_shape=[
        jax.ShapeDtypeStruct((N, D), jnp.float32),
        jax.ShapeDtypeStruct((N, DH), jnp.float32),
        jax.ShapeDtypeStruct((N, DH), jnp.float32),
    ],
)


def _tc_fin_body(sa_ref, sb_ref, hwsa_ref, hwsb_ref, dinv_ref, hprev_ref,
                 bc_ref, g_ref, bl_ref, wo_ref, bo_ref, out_ref):
    dinv = dinv_ref[...][:, :1]
    h2 = _layer_tail(sa_ref[...], sb_ref[...], hwsa_ref[...], hwsb_ref[...],
                     dinv, hprev_ref[...], bc_ref[...], g_ref[...], bl_ref[...])
    out_ref[...] = lax.dot_general(
        h2, wo_ref[...], _CONTRACT, preferred_element_type=jnp.float32
    ) + bo_ref[...]


_tc_fin = pl.pallas_call(
    _tc_fin_body,
    grid=(GRID,),
    in_specs=[
        pl.BlockSpec((BN, DH), lambda i: (i, 0)),
        pl.BlockSpec((BN, DH), lambda i: (i, 0)),
        pl.BlockSpec((BN, DH), lambda i: (i, 0)),
        pl.BlockSpec((BN, DH), lambda i: (i, 0)),
        pl.BlockSpec((BN, DH), lambda i: (i, 0)),
        pl.BlockSpec((BN, D), lambda i: (i, 0)),
        pl.BlockSpec((1, D), lambda i: (0, 0)),
        pl.BlockSpec((1, D), lambda i: (0, 0)),
        pl.BlockSpec((1, D), lambda i: (0, 0)),
        pl.BlockSpec((D, D), lambda i: (0, 0)),
        pl.BlockSpec((1, D), lambda i: (0, 0)),
    ],
    out_specs=pl.BlockSpec((BN, D), lambda i: (i, 0)),
    out_shape=jax.ShapeDtypeStruct((N, D), jnp.float32),
)


def kernel(x, edge_index, W_in, b_in, W_c1, b_c1, g_ln1, b_ln1,
           W_c2, b_c2, g_ln2, b_ln2, W_out, b_out):
    src = edge_index[0]
    dst = edge_index[1]

    degp = _deg_pass_built()(dst).reshape(2, N, DH)
    h0, dinv, hws1a, hws1b = _tc_pre(
        x, W_in, b_in.reshape(1, D), W_c1, degp)
    s1a, s1b = _row_pass_built()(hws1a, hws1b, src, dst)
    h1, hws2a, hws2b = _tc_mid(
        s1a, s1b, hws1a, hws1b, dinv, h0,
        b_c1.reshape(1, D), g_ln1.reshape(1, D), b_ln1.reshape(1, D), W_c2)
    s2a, s2b = _row_pass_built()(hws2a, hws2b, src, dst)
    out = _tc_fin(
        s2a, s2b, hws2a, hws2b, dinv, h1,
        b_c2.reshape(1, D), g_ln2.reshape(1, D), b_ln2.reshape(1, D),
        W_out, b_out.reshape(1, D))
    return out


# speedup vs baseline: 17.7780x; 1.0005x over previous
import functools

import jax
import jax.numpy as jnp
from jax import lax
from jax.experimental import pallas as pl
from jax.experimental.pallas import tpu as pltpu
from jax.experimental.pallas import tpu_sc as plsc

N = 10000
E = 160000
D = 256
DH = 128
EPS = 1e-5

NC = 2
NS = 16
RQ = 624
EPSC = E // NS
CH = 128
NFULL = EPSC // CH
TAIL = EPSC - NFULL * CH
EPW = E // (NC * NS)
NFULL_D = EPW // CH
TAIL_D = EPW - NFULL_D * CH

@functools.cache
def _get_mesh():
    return plsc.VectorSubcoreMesh(
        core_axis_name="c", subcore_axis_name="s", num_cores=NC, num_subcores=NS
    )


@functools.cache
def _deg_pass_built():
    return pl.kernel(
        _deg_body,
        mesh=_get_mesh(),
        out_type=jax.ShapeDtypeStruct((2 * N, DH), jnp.float32),
        scratch_types=[
            pltpu.VMEM((CH,), jnp.int32),
            pltpu.VMEM((TAIL_D,), jnp.int32),
            pltpu.VMEM((CH, DH), jnp.float32),
            pltpu.VMEM((TAIL_D, DH), jnp.float32),
            pltpu.VMEM((16, DH), jnp.float32),
            pltpu.VMEM_SHARED((N, DH), jnp.float32),
        ],
    )


def _deg_body(dst_hbm, out_hbm, dst_v, dst_t, ones_v, ones_t, zbuf, acc):
    c = lax.axis_index("c")
    s = lax.axis_index("s")

    def zfill(r, carry):
        for j in range(DH // 16):
            zbuf[r, pl.ds(j * 16, 16)] = jnp.zeros((16,), jnp.float32)
        return carry

    lax.fori_loop(0, 16, zfill, 0)

    def ofill(r, carry):
        for j in range(DH // 16):
            ones_v[r, pl.ds(j * 16, 16)] = jnp.full((16,), 1.0, jnp.float32)
        return carry

    lax.fori_loop(0, CH, ofill, 0)
    for r in range(TAIL_D):
        for j in range(DH // 16):
            ones_t[r, pl.ds(j * 16, 16)] = jnp.full((16,), 1.0, jnp.float32)

    row0 = pl.multiple_of(s * RQ, 8)
    nsteps = jnp.where(s == NS - 1, (N - (NS - 1) * RQ) // 16, RQ // 16)

    def zstep(t, carry):
        pltpu.sync_copy(zbuf, acc.at[pl.ds(row0 + t * 16, 16)])
        return carry

    lax.fori_loop(0, nsteps, zstep, 0)
    plsc.subcore_barrier()

    base0 = (s * NC + c) * EPW

    def chunk(t, carry):
        base = pl.multiple_of(base0 + t * CH, 8)
        pltpu.sync_copy(dst_hbm.at[pl.ds(base, CH)], dst_v)
        pltpu.sync_copy(ones_v, acc.at[dst_v], add=True)
        return carry

    lax.fori_loop(0, NFULL_D, chunk, 0)
    base = pl.multiple_of(base0 + NFULL_D * CH, 8)
    pltpu.sync_copy(dst_hbm.at[pl.ds(base, TAIL_D)], dst_t)
    pltpu.sync_copy(ones_t, acc.at[dst_t], add=True)
    plsc.subcore_barrier()
    pltpu.sync_copy(acc.at[pl.ds(row0, RQ)], out_hbm.at[pl.ds(c * N + row0, RQ)])

    @pl.when(s == NS - 1)
    def _():
        extra = NS * RQ
        nex = N - extra
        pltpu.sync_copy(acc.at[pl.ds(extra, nex)],
                        out_hbm.at[pl.ds(c * N + extra, nex)])


CHK = 80
NCHS = EPSC // CHK
NTRI = (NCHS - 2) // 3


@functools.cache
def _row_pass_built():
    return pl.kernel(
        _row_body,
        mesh=_get_mesh(),
        out_type=[
            jax.ShapeDtypeStruct((N, DH), jnp.float32),
            jax.ShapeDtypeStruct((N, DH), jnp.float32),
        ],
        scratch_types=[
            pltpu.VMEM((EPSC,), jnp.int32),
            pltpu.VMEM((CHK,), jnp.int32),
            pltpu.VMEM((CHK,), jnp.int32),
            pltpu.VMEM((CHK,), jnp.int32),
            pltpu.VMEM((CHK, DH), jnp.float32),
            pltpu.VMEM((CHK, DH), jnp.float32),
            pltpu.VMEM((CHK, DH), jnp.float32),
            pltpu.VMEM((16, DH), jnp.float32),
            pltpu.VMEM_SHARED((N, DH), jnp.float32),
            pltpu.SemaphoreType.DMA,
            pltpu.SemaphoreType.DMA,
            pltpu.SemaphoreType.DMA,
            pltpu.SemaphoreType.DMA,
            pltpu.SemaphoreType.DMA,
            pltpu.SemaphoreType.DMA,
        ],
    )


def _row_body(hws_a, hws_b, src_hbm, dst_hbm, out_a, out_b,
              src_v, dst_p0, dst_p1, dst_p2, rows_p0, rows_p1, rows_p2,
              zbuf, acc,
              sem_g0, sem_g1, sem_g2, sem_s0, sem_s1, sem_s2):
    c = lax.axis_index("c")
    s = lax.axis_index("s")

    def zfill(r, carry):
        for j in range(DH // 16):
            zbuf[r, pl.ds(j * 16, 16)] = jnp.zeros((16,), jnp.float32)
        return carry

    lax.fori_loop(0, 16, zfill, 0)
    row0 = pl.multiple_of(s * RQ, 8)
    nsteps = jnp.where(s == NS - 1, (N - (NS - 1) * RQ) // 16, RQ // 16)

    def zstep(t, carry):
        pltpu.sync_copy(zbuf, acc.at[pl.ds(row0 + t * 16, 16)])
        return carry

    lax.fori_loop(0, nsteps, zstep, 0)

    base_e = pl.multiple_of(s * EPSC, 8)
    pltpu.sync_copy(src_hbm.at[pl.ds(base_e, EPSC)], src_v)
    plsc.subcore_barrier()

    def run(hws_hbm, out_hbm):
        def dma_g(t, dstb, rows, semg, issue):
            off = pl.multiple_of(base_e + t * CHK, 8)
            idx = src_v.at[pl.ds(pl.multiple_of(t * CHK, 8), CHK)]
            if issue:
                pltpu.async_copy(dst_hbm.at[pl.ds(off, CHK)], dstb, semg)
                pltpu.async_copy(hws_hbm.at[idx], rows, semg)
            else:
                pltpu.make_async_copy(dst_hbm.at[pl.ds(off, CHK)], dstb, semg).wait()
                pltpu.make_async_copy(hws_hbm.at[idx], rows, semg).wait()

        def dma_s(dstb, rows, sems, issue):
            if issue:
                pltpu.async_copy(rows, acc.at[dstb], sems, add=True)
            else:
                pltpu.make_async_copy(rows, acc.at[dstb], sems).wait()

        bufs = ((dst_p0, rows_p0, sem_g0, sem_s0),
                (dst_p1, rows_p1, sem_g1, sem_s1),
                (dst_p2, rows_p2, sem_g2, sem_s2))

        def step(t, b, bprev):
            dma_g(t, b[0], b[1], b[2], False)
            dma_s(bprev[0], bprev[1], bprev[3], False)
            dma_s(b[0], b[1], b[3], True)

            @pl.when(t + 2 < NCHS)
            def _():
                dma_g(t + 2, bprev[0], bprev[1], bprev[2], True)

        dma_g(0, dst_p0, rows_p0, sem_g0, True)
        dma_g(1, dst_p1, rows_p1, sem_g1, True)
        dma_g(0, dst_p0, rows_p0, sem_g0, False)
        dma_s(dst_p0, rows_p0, sem_s0, True)
        dma_g(2, dst_p2, rows_p2, sem_g2, True)
        step(1, bufs[1], bufs[0])

        def tri(k, carry):
            t = 3 * k + 2
            step(t, bufs[2], bufs[1])
            step(t + 1, bufs[0], bufs[2])
            step(t + 2, bufs[1], bufs[0])
            return carry

        lax.fori_loop(0, NTRI, tri, 0)
        dma_s(dst_p1, rows_p1, sem_s1, False)
        plsc.subcore_barrier()
        pltpu.sync_copy(acc.at[pl.ds(row0, RQ)], out_hbm.at[pl.ds(row0, RQ)])

        @pl.when(s == NS - 1)
        def _():
            extra = NS * RQ
            nex = N - extra
            pltpu.sync_copy(acc.at[pl.ds(extra, nex)],
                            out_hbm.at[pl.ds(extra, nex)])

    @pl.when(c == 0)
    def _():
        run(hws_a, out_a)

    @pl.when(c == 1)
    def _():
        run(hws_b, out_b)


BN = 1000
GRID = N // BN
_CONTRACT = (((1,), (1,)), ((), ()))


def _mm(x, w):
    return lax.dot_general(x.astype(jnp.bfloat16), w.astype(jnp.bfloat16),
                           _CONTRACT, preferred_element_type=jnp.float32)


def _tc_pre_body(x_ref, win_ref, bin_ref, w1_ref, degp_ref,
                 h0_ref, dinv_ref, hwsa_ref, hwsb_ref):
    x = x_ref[...]
    h0 = jnp.maximum(_mm(x, win_ref[...]) + bin_ref[...], 0.0)
    dp = degp_ref[...]
    deg = dp[0][:, :1] + dp[1][:, :1] + 1.0
    dinv = lax.rsqrt(deg)
    hw = _mm(h0, w1_ref[...])
    hws = hw * dinv
    h0_ref[...] = h0
    dinv_ref[...] = jnp.broadcast_to(dinv, (BN, DH))
    hwsa_ref[...] = hws[:, :DH]
    hwsb_ref[...] = hws[:, DH:]


_tc_pre = pl.pallas_call(
    _tc_pre_body,
    grid=(GRID,),
    in_specs=[
        pl.BlockSpec((BN, D), lambda i: (i, 0)),
        pl.BlockSpec((D, D), lambda i: (0, 0)),
        pl.BlockSpec((1, D), lambda i: (0, 0)),
        pl.BlockSpec((D, D), lambda i: (0, 0)),
        pl.BlockSpec((2, BN, DH), lambda i: (0, i, 0)),
    ],
    out_specs=[
        pl.BlockSpec((BN, D), lambda i: (i, 0)),
        pl.BlockSpec((BN, DH), lambda i: (i, 0)),
        pl.BlockSpec((BN, DH), lambda i: (i, 0)),
        pl.BlockSpec((BN, DH), lambda i: (i, 0)),
    ],
    out_shape=[
        jax.ShapeDtypeStruct((N, D), jnp.float32),
        jax.ShapeDtypeStruct((N, DH), jnp.float32),
        jax.ShapeDtypeStruct((N, DH), jnp.float32),
        jax.ShapeDtypeStruct((N, DH), jnp.float32),
    ],
)


def _layer_tail(sa, sb, hwsa, hwsb, dinv, hprev, bc, g, bl):
    conv = jnp.concatenate([sa + hwsa, sb + hwsb], axis=1) * dinv + bc
    t = hprev + jnp.maximum(conv, 0.0)
    mu = jnp.mean(t, axis=1, keepdims=True)
    var = jnp.mean((t - mu) ** 2, axis=1, keepdims=True)
    return (t - mu) * lax.rsqrt(var + EPS) * g + bl


def _tc_mid_body(sa_ref, sb_ref, hwsa_ref, hwsb_ref, dinv_ref, hprev_ref,
                 bc_ref, g_ref, bl_ref, w2_ref,
                 h1_ref, h2a_ref, h2b_ref):
    dinv = dinv_ref[...][:, :1]
    h1 = _layer_tail(sa_ref[...], sb_ref[...], hwsa_ref[...], hwsb_ref[...],
                     dinv, hprev_ref[...], bc_ref[...], g_ref[...], bl_ref[...])
    h1_ref[...] = h1
    hw2 = _mm(h1, w2_ref[...])
    hws2 = hw2 * dinv
    h2a_ref[...] = hws2[:, :DH]
    h2b_ref[...] = hws2[:, DH:]


_tc_mid = pl.pallas_call(
    _tc_mid_body,
    grid=(GRID,),
    in_specs=[
        pl.BlockSpec((BN, DH), lambda i: (i, 0)),
        pl.BlockSpec((BN, DH), lambda i: (i, 0)),
        pl.BlockSpec((BN, DH), lambda i: (i, 0)),
        pl.BlockSpec((BN, DH), lambda i: (i, 0)),
        pl.BlockSpec((BN, DH), lambda i: (i, 0)),
        pl.BlockSpec((BN, D), lambda i: (i, 0)),
        pl.BlockSpec((1, D), lambda i: (0, 0)),
        pl.BlockSpec((1, D), lambda i: (0, 0)),
        pl.BlockSpec((1, D), lambda i: (0, 0)),
        pl.BlockSpec((D, D), lambda i: (0, 0)),
    ],
    out_specs=[
        pl.BlockSpec((BN, D), lambda i: (i, 0)),
        pl.BlockSpec((BN, DH), lambda i: (i, 0)),
        pl.BlockSpec((BN, DH), lambda i: (i, 0)),
    ],
    out_shape=[
        jax.ShapeDtypeStruct((N, D), jnp.float32),
        jax.ShapeDtypeStruct((N, DH), jnp.float32),
        jax.ShapeDtypeStruct((N, DH), jnp.float32),
    ],
)


def _tc_fin_body(sa_ref, sb_ref, hwsa_ref, hwsb_ref, dinv_ref, hprev_ref,
                 bc_ref, g_ref, bl_ref, wo_ref, bo_ref, out_ref):
    dinv = dinv_ref[...][:, :1]
    h2 = _layer_tail(sa_ref[...], sb_ref[...], hwsa_ref[...], hwsb_ref[...],
                     dinv, hprev_ref[...], bc_ref[...], g_ref[...], bl_ref[...])
    out_ref[...] = _mm(h2, wo_ref[...]) + bo_ref[...]


_tc_fin = pl.pallas_call(
    _tc_fin_body,
    grid=(GRID,),
    in_specs=[
        pl.BlockSpec((BN, DH), lambda i: (i, 0)),
        pl.BlockSpec((BN, DH), lambda i: (i, 0)),
        pl.BlockSpec((BN, DH), lambda i: (i, 0)),
        pl.BlockSpec((BN, DH), lambda i: (i, 0)),
        pl.BlockSpec((BN, DH), lambda i: (i, 0)),
        pl.BlockSpec((BN, D), lambda i: (i, 0)),
        pl.BlockSpec((1, D), lambda i: (0, 0)),
        pl.BlockSpec((1, D), lambda i: (0, 0)),
        pl.BlockSpec((1, D), lambda i: (0, 0)),
        pl.BlockSpec((D, D), lambda i: (0, 0)),
        pl.BlockSpec((1, D), lambda i: (0, 0)),
    ],
    out_specs=pl.BlockSpec((BN, D), lambda i: (i, 0)),
    out_shape=jax.ShapeDtypeStruct((N, D), jnp.float32),
)


def kernel(x, edge_index, W_in, b_in, W_c1, b_c1, g_ln1, b_ln1,
           W_c2, b_c2, g_ln2, b_ln2, W_out, b_out):
    src = edge_index[0]
    dst = edge_index[1]

    degp = _deg_pass_built()(dst).reshape(2, N, DH)
    h0, dinv, hws1a, hws1b = _tc_pre(
        x, W_in, b_in.reshape(1, D), W_c1, degp)
    s1a, s1b = _row_pass_built()(hws1a, hws1b, src, dst)
    h1, hws2a, hws2b = _tc_mid(
        s1a, s1b, hws1a, hws1b, dinv, h0,
        b_c1.reshape(1, D), g_ln1.reshape(1, D), b_ln1.reshape(1, D), W_c2)
    s2a, s2b = _row_pass_built()(hws2a, hws2b, src, dst)
    out = _tc_fin(
        s2a, s2b, hws2a, hws2b, dinv, h1,
        b_c2.reshape(1, D), g_ln2.reshape(1, D), b_ln2.reshape(1, D),
        W_out, b_out.reshape(1, D))
    return out


# speedup vs baseline: 18.1995x; 1.0237x over previous
import functools

import jax
import jax.numpy as jnp
from jax import lax
from jax.experimental import pallas as pl
from jax.experimental.pallas import tpu as pltpu
from jax.experimental.pallas import tpu_sc as plsc

N = 10000
E = 160000
D = 256
DH = 128
EPS = 1e-5

NC = 2
NS = 16
RQ = 624
EPSC = E // NS
CH = 128
NFULL = EPSC // CH
TAIL = EPSC - NFULL * CH
EPW = E // (NC * NS)
NFULL_D = EPW // CH
TAIL_D = EPW - NFULL_D * CH
DEGW = 128

@functools.cache
def _get_mesh():
    return plsc.VectorSubcoreMesh(
        core_axis_name="c", subcore_axis_name="s", num_cores=NC, num_subcores=NS
    )


@functools.cache
def _deg_pass_built():
    return pl.kernel(
        _deg_body,
        mesh=_get_mesh(),
        out_type=jax.ShapeDtypeStruct((2 * N, DEGW), jnp.float32),
        scratch_types=[
            pltpu.VMEM((CH,), jnp.int32),
            pltpu.VMEM((TAIL_D,), jnp.int32),
            pltpu.VMEM((CH, DEGW), jnp.float32),
            pltpu.VMEM((TAIL_D, DEGW), jnp.float32),
            pltpu.VMEM((16, DEGW), jnp.float32),
            pltpu.VMEM_SHARED((N, DEGW), jnp.float32),
        ],
    )


def _deg_body(dst_hbm, out_hbm, dst_v, dst_t, ones_v, ones_t, zbuf, acc):
    c = lax.axis_index("c")
    s = lax.axis_index("s")

    def zfill(r, carry):
        for j in range(DEGW // 16):
            zbuf[r, pl.ds(j * 16, 16)] = jnp.zeros((16,), jnp.float32)
        return carry

    lax.fori_loop(0, 16, zfill, 0)

    def ofill(r, carry):
        for j in range(DEGW // 16):
            ones_v[r, pl.ds(j * 16, 16)] = jnp.full((16,), 1.0, jnp.float32)
        return carry

    lax.fori_loop(0, CH, ofill, 0)
    for r in range(TAIL_D):
        for j in range(DEGW // 16):
            ones_t[r, pl.ds(j * 16, 16)] = jnp.full((16,), 1.0, jnp.float32)

    row0 = pl.multiple_of(s * RQ, 8)
    nsteps = jnp.where(s == NS - 1, (N - (NS - 1) * RQ) // 16, RQ // 16)

    def zstep(t, carry):
        pltpu.sync_copy(zbuf, acc.at[pl.ds(row0 + t * 16, 16)])
        return carry

    lax.fori_loop(0, nsteps, zstep, 0)
    plsc.subcore_barrier()

    base0 = (s * NC + c) * EPW

    def chunk(t, carry):
        base = pl.multiple_of(base0 + t * CH, 8)
        pltpu.sync_copy(dst_hbm.at[pl.ds(base, CH)], dst_v)
        pltpu.sync_copy(ones_v, acc.at[dst_v], add=True)
        return carry

    lax.fori_loop(0, NFULL_D, chunk, 0)
    base = pl.multiple_of(base0 + NFULL_D * CH, 8)
    pltpu.sync_copy(dst_hbm.at[pl.ds(base, TAIL_D)], dst_t)
    pltpu.sync_copy(ones_t, acc.at[dst_t], add=True)
    plsc.subcore_barrier()
    pltpu.sync_copy(acc.at[pl.ds(row0, RQ)], out_hbm.at[pl.ds(c * N + row0, RQ)])

    @pl.when(s == NS - 1)
    def _():
        extra = NS * RQ
        nex = N - extra
        pltpu.sync_copy(acc.at[pl.ds(extra, nex)],
                        out_hbm.at[pl.ds(c * N + extra, nex)])


CHK = 80
NCHS = EPSC // CHK
NTRI = (NCHS - 2) // 3


@functools.cache
def _row_pass_built():
    return pl.kernel(
        _row_body,
        mesh=_get_mesh(),
        out_type=[
            jax.ShapeDtypeStruct((N, DH), jnp.float32),
            jax.ShapeDtypeStruct((N, DH), jnp.float32),
        ],
        scratch_types=[
            pltpu.VMEM((EPSC,), jnp.int32),
            pltpu.VMEM((CHK,), jnp.int32),
            pltpu.VMEM((CHK,), jnp.int32),
            pltpu.VMEM((CHK,), jnp.int32),
            pltpu.VMEM((CHK, DH), jnp.float32),
            pltpu.VMEM((CHK, DH), jnp.float32),
            pltpu.VMEM((CHK, DH), jnp.float32),
            pltpu.VMEM((16, DH), jnp.float32),
            pltpu.VMEM_SHARED((N, DH), jnp.float32),
            pltpu.SemaphoreType.DMA,
            pltpu.SemaphoreType.DMA,
            pltpu.SemaphoreType.DMA,
            pltpu.SemaphoreType.DMA,
            pltpu.SemaphoreType.DMA,
            pltpu.SemaphoreType.DMA,
        ],
    )


def _row_body(hws_a, hws_b, src_hbm, dst_hbm, out_a, out_b,
              src_v, dst_p0, dst_p1, dst_p2, rows_p0, rows_p1, rows_p2,
              zbuf, acc,
              sem_g0, sem_g1, sem_g2, sem_s0, sem_s1, sem_s2):
    c = lax.axis_index("c")
    s = lax.axis_index("s")

    def zfill(r, carry):
        for j in range(DH // 16):
            zbuf[r, pl.ds(j * 16, 16)] = jnp.zeros((16,), jnp.float32)
        return carry

    lax.fori_loop(0, 16, zfill, 0)
    row0 = pl.multiple_of(s * RQ, 8)
    nsteps = jnp.where(s == NS - 1, (N - (NS - 1) * RQ) // 16, RQ // 16)

    def zstep(t, carry):
        pltpu.sync_copy(zbuf, acc.at[pl.ds(row0 + t * 16, 16)])
        return carry

    lax.fori_loop(0, nsteps, zstep, 0)

    base_e = pl.multiple_of(s * EPSC, 8)
    pltpu.sync_copy(src_hbm.at[pl.ds(base_e, EPSC)], src_v)
    plsc.subcore_barrier()

    def run(hws_hbm, out_hbm):
        def dma_g(t, dstb, rows, semg, issue):
            off = pl.multiple_of(base_e + t * CHK, 8)
            idx = src_v.at[pl.ds(pl.multiple_of(t * CHK, 8), CHK)]
            if issue:
                pltpu.async_copy(dst_hbm.at[pl.ds(off, CHK)], dstb, semg)
                pltpu.async_copy(hws_hbm.at[idx], rows, semg)
            else:
                pltpu.make_async_copy(dst_hbm.at[pl.ds(off, CHK)], dstb, semg).wait()
                pltpu.make_async_copy(hws_hbm.at[idx], rows, semg).wait()

        def dma_s(dstb, rows, sems, issue):
            if issue:
                pltpu.async_copy(rows, acc.at[dstb], sems, add=True)
            else:
                pltpu.make_async_copy(rows, acc.at[dstb], sems).wait()

        bufs = ((dst_p0, rows_p0, sem_g0, sem_s0),
                (dst_p1, rows_p1, sem_g1, sem_s1),
                (dst_p2, rows_p2, sem_g2, sem_s2))

        def step(t, b, bprev):
            dma_g(t, b[0], b[1], b[2], False)
            dma_s(bprev[0], bprev[1], bprev[3], False)
            dma_s(b[0], b[1], b[3], True)

            @pl.when(t + 2 < NCHS)
            def _():
                dma_g(t + 2, bprev[0], bprev[1], bprev[2], True)

        dma_g(0, dst_p0, rows_p0, sem_g0, True)
        dma_g(1, dst_p1, rows_p1, sem_g1, True)
        dma_g(0, dst_p0, rows_p0, sem_g0, False)
        dma_s(dst_p0, rows_p0, sem_s0, True)
        dma_g(2, dst_p2, rows_p2, sem_g2, True)
        step(1, bufs[1], bufs[0])

        def tri(k, carry):
            t = 3 * k + 2
            step(t, bufs[2], bufs[1])
            step(t + 1, bufs[0], bufs[2])
            step(t + 2, bufs[1], bufs[0])
            return carry

        lax.fori_loop(0, NTRI, tri, 0)
        dma_s(dst_p1, rows_p1, sem_s1, False)
        plsc.subcore_barrier()
        pltpu.sync_copy(acc.at[pl.ds(row0, RQ)], out_hbm.at[pl.ds(row0, RQ)])

        @pl.when(s == NS - 1)
        def _():
            extra = NS * RQ
            nex = N - extra
            pltpu.sync_copy(acc.at[pl.ds(extra, nex)],
                            out_hbm.at[pl.ds(extra, nex)])

    @pl.when(c == 0)
    def _():
        run(hws_a, out_a)

    @pl.when(c == 1)
    def _():
        run(hws_b, out_b)


BN = 1000
GRID = N // BN
_CONTRACT = (((1,), (1,)), ((), ()))


def _mm(x, w):
    return lax.dot_general(x, w, _CONTRACT, preferred_element_type=jnp.float32)


def _tc_h0_body(x_ref, win_ref, bin_ref, h0_ref):
    h0_ref[...] = jnp.maximum(_mm(x_ref[...], win_ref[...]) + bin_ref[...], 0.0)


_tc_h0 = pl.pallas_call(
    _tc_h0_body,
    grid=(GRID,),
    in_specs=[
        pl.BlockSpec((BN, D), lambda i: (i, 0)),
        pl.BlockSpec((D, D), lambda i: (0, 0)),
        pl.BlockSpec((1, D), lambda i: (0, 0)),
    ],
    out_specs=pl.BlockSpec((BN, D), lambda i: (i, 0)),
    out_shape=jax.ShapeDtypeStruct((N, D), jnp.float32),
)


def _tc_scale_body(h0_ref, w1_ref, degp_ref, dinv_ref, hwsa_ref, hwsb_ref):
    dp = degp_ref[...]
    deg = dp[0][:, :1] + dp[1][:, :1] + 1.0
    dinv = lax.rsqrt(deg)
    hw = _mm(h0_ref[...], w1_ref[...])
    hws = hw * dinv
    dinv_ref[...] = jnp.broadcast_to(dinv, (BN, DH))
    hwsa_ref[...] = hws[:, :DH]
    hwsb_ref[...] = hws[:, DH:]


_tc_scale = pl.pallas_call(
    _tc_scale_body,
    grid=(GRID,),
    in_specs=[
        pl.BlockSpec((BN, D), lambda i: (i, 0)),
        pl.BlockSpec((D, D), lambda i: (0, 0)),
        pl.BlockSpec((2, BN, DEGW), lambda i: (0, i, 0)),
    ],
    out_specs=[
        pl.BlockSpec((BN, DH), lambda i: (i, 0)),
        pl.BlockSpec((BN, DH), lambda i: (i, 0)),
        pl.BlockSpec((BN, DH), lambda i: (i, 0)),
    ],
    out_shape=[
        jax.ShapeDtypeStruct((N, DH), jnp.float32),
        jax.ShapeDtypeStruct((N, DH), jnp.float32),
        jax.ShapeDtypeStruct((N, DH), jnp.float32),
    ],
)


def _layer_tail(sa, sb, hwsa, hwsb, dinv, hprev, bc, g, bl):
    conv = jnp.concatenate([sa + hwsa, sb + hwsb], axis=1) * dinv + bc
    t = hprev + jnp.maximum(conv, 0.0)
    mu = jnp.mean(t, axis=1, keepdims=True)
    var = jnp.mean((t - mu) ** 2, axis=1, keepdims=True)
    return (t - mu) * lax.rsqrt(var + EPS) * g + bl


def _tc_mid_body(sa_ref, sb_ref, hwsa_ref, hwsb_ref, dinv_ref, hprev_ref,
                 bc_ref, g_ref, bl_ref, w2_ref,
                 h1_ref, h2a_ref, h2b_ref):
    dinv = dinv_ref[...][:, :1]
    h1 = _layer_tail(sa_ref[...], sb_ref[...], hwsa_ref[...], hwsb_ref[...],
                     dinv, hprev_ref[...], bc_ref[...], g_ref[...], bl_ref[...])
    h1_ref[...] = h1
    hw2 = _mm(h1, w2_ref[...])
    hws2 = hw2 * dinv
    h2a_ref[...] = hws2[:, :DH]
    h2b_ref[...] = hws2[:, DH:]


_tc_mid = pl.pallas_call(
    _tc_mid_body,
    grid=(GRID,),
    in_specs=[
        pl.BlockSpec((BN, DH), lambda i: (i, 0)),
        pl.BlockSpec((BN, DH), lambda i: (i, 0)),
        pl.BlockSpec((BN, DH), lambda i: (i, 0)),
        pl.BlockSpec((BN, DH), lambda i: (i, 0)),
        pl.BlockSpec((BN, DH), lambda i: (i, 0)),
        pl.BlockSpec((BN, D), lambda i: (i, 0)),
        pl.BlockSpec((1, D), lambda i: (0, 0)),
        pl.BlockSpec((1, D), lambda i: (0, 0)),
        pl.BlockSpec((1, D), lambda i: (0, 0)),
        pl.BlockSpec((D, D), lambda i: (0, 0)),
    ],
    out_specs=[
        pl.BlockSpec((BN, D), lambda i: (i, 0)),
        pl.BlockSpec((BN, DH), lambda i: (i, 0)),
        pl.BlockSpec((BN, DH), lambda i: (i, 0)),
    ],
    out_shape=[
        jax.ShapeDtypeStruct((N, D), jnp.float32),
        jax.ShapeDtypeStruct((N, DH), jnp.float32),
        jax.ShapeDtypeStruct((N, DH), jnp.float32),
    ],
)


def _tc_fin_body(sa_ref, sb_ref, hwsa_ref, hwsb_ref, dinv_ref, hprev_ref,
                 bc_ref, g_ref, bl_ref, wo_ref, bo_ref, out_ref):
    dinv = dinv_ref[...][:, :1]
    h2 = _layer_tail(sa_ref[...], sb_ref[...], hwsa_ref[...], hwsb_ref[...],
                     dinv, hprev_ref[...], bc_ref[...], g_ref[...], bl_ref[...])
    out_ref[...] = _mm(h2, wo_ref[...]) + bo_ref[...]


_tc_fin = pl.pallas_call(
    _tc_fin_body,
    grid=(GRID,),
    in_specs=[
        pl.BlockSpec((BN, DH), lambda i: (i, 0)),
        pl.BlockSpec((BN, DH), lambda i: (i, 0)),
        pl.BlockSpec((BN, DH), lambda i: (i, 0)),
        pl.BlockSpec((BN, DH), lambda i: (i, 0)),
        pl.BlockSpec((BN, DH), lambda i: (i, 0)),
        pl.BlockSpec((BN, D), lambda i: (i, 0)),
        pl.BlockSpec((1, D), lambda i: (0, 0)),
        pl.BlockSpec((1, D), lambda i: (0, 0)),
        pl.BlockSpec((1, D), lambda i: (0, 0)),
        pl.BlockSpec((D, D), lambda i: (0, 0)),
        pl.BlockSpec((1, D), lambda i: (0, 0)),
    ],
    out_specs=pl.BlockSpec((BN, D), lambda i: (i, 0)),
    out_shape=jax.ShapeDtypeStruct((N, D), jnp.float32),
)


def kernel(x, edge_index, W_in, b_in, W_c1, b_c1, g_ln1, b_ln1,
           W_c2, b_c2, g_ln2, b_ln2, W_out, b_out):
    src = edge_index[0]
    dst = edge_index[1]

    degp = _deg_pass_built()(dst).reshape(2, N, DEGW)
    h0 = _tc_h0(x, W_in, b_in.reshape(1, D))
    dinv, hws1a, hws1b = _tc_scale(h0, W_c1, degp)
    s1a, s1b = _row_pass_built()(hws1a, hws1b, src, dst)
    h1, hws2a, hws2b = _tc_mid(
        s1a, s1b, hws1a, hws1b, dinv, h0,
        b_c1.reshape(1, D), g_ln1.reshape(1, D), b_ln1.reshape(1, D), W_c2)
    s2a, s2b = _row_pass_built()(hws2a, hws2b, src, dst)
    out = _tc_fin(
        s2a, s2b, hws2a, hws2b, dinv, h1,
        b_c2.reshape(1, D), g_ln2.reshape(1, D), b_ln2.reshape(1, D),
        W_out, b_out.reshape(1, D))
    return out


# speedup vs baseline: 18.8167x; 1.0339x over previous
import functools

import jax
import jax.numpy as jnp
from jax import lax
from jax.experimental import pallas as pl
from jax.experimental.pallas import tpu as pltpu
from jax.experimental.pallas import tpu_sc as plsc

N = 10000
E = 160000
D = 256
DH = 128
EPS = 1e-5

NC = 2
NS = 16
RQ = 624
EPSC = E // NS
CH = 128
NFULL = EPSC // CH
TAIL = EPSC - NFULL * CH
EPW = E // (NC * NS)
NFULL_D = EPW // CH
TAIL_D = EPW - NFULL_D * CH
DEGW = 128

@functools.cache
def _get_mesh():
    return plsc.VectorSubcoreMesh(
        core_axis_name="c", subcore_axis_name="s", num_cores=NC, num_subcores=NS
    )


@functools.cache
def _deg_pass_built():
    return pl.kernel(
        _deg_body,
        mesh=_get_mesh(),
        out_type=jax.ShapeDtypeStruct((2 * N, DEGW), jnp.float32),
        scratch_types=[
            pltpu.VMEM((CH,), jnp.int32),
            pltpu.VMEM((CH,), jnp.int32),
            pltpu.VMEM((TAIL_D,), jnp.int32),
            pltpu.VMEM((CH, DEGW), jnp.float32),
            pltpu.VMEM((TAIL_D, DEGW), jnp.float32),
            pltpu.VMEM((16, DEGW), jnp.float32),
            pltpu.VMEM_SHARED((N, DEGW), jnp.float32),
            pltpu.SemaphoreType.DMA,
            pltpu.SemaphoreType.DMA,
            pltpu.SemaphoreType.DMA,
            pltpu.SemaphoreType.DMA,
        ],
    )


def _deg_body(dst_hbm, out_hbm, dst_pa, dst_pb, dst_t, ones_v, ones_t, zbuf, acc,
              sem_da, sem_db, sem_sa, sem_sb):
    c = lax.axis_index("c")
    s = lax.axis_index("s")

    def zfill(r, carry):
        for j in range(DEGW // 16):
            zbuf[r, pl.ds(j * 16, 16)] = jnp.zeros((16,), jnp.float32)
        return carry

    lax.fori_loop(0, 16, zfill, 0)

    def ofill(r, carry):
        for j in range(DEGW // 16):
            ones_v[r, pl.ds(j * 16, 16)] = jnp.full((16,), 1.0, jnp.float32)
        return carry

    lax.fori_loop(0, CH, ofill, 0)
    for r in range(TAIL_D):
        for j in range(DEGW // 16):
            ones_t[r, pl.ds(j * 16, 16)] = jnp.full((16,), 1.0, jnp.float32)

    row0 = pl.multiple_of(s * RQ, 8)
    nsteps = jnp.where(s == NS - 1, (N - (NS - 1) * RQ) // 16, RQ // 16)

    def zstep(t, carry):
        pltpu.sync_copy(zbuf, acc.at[pl.ds(row0 + t * 16, 16)])
        return carry

    lax.fori_loop(0, nsteps, zstep, 0)
    plsc.subcore_barrier()

    base0 = (s * NC + c) * EPW

    def dma_d(t, dstb, semd, issue):
        base = pl.multiple_of(base0 + t * CH, 8)
        if issue:
            pltpu.async_copy(dst_hbm.at[pl.ds(base, CH)], dstb, semd)
        else:
            pltpu.make_async_copy(dst_hbm.at[pl.ds(base, CH)], dstb, semd).wait()

    def dma_s(dstb, sems, issue):
        if issue:
            pltpu.async_copy(ones_v, acc.at[dstb], sems, add=True)
        else:
            pltpu.make_async_copy(ones_v, acc.at[dstb], sems).wait()

    dma_d(0, dst_pa, sem_da, True)
    dma_d(0, dst_pa, sem_da, False)
    dma_s(dst_pa, sem_sa, True)
    dma_d(1, dst_pb, sem_db, True)

    def pair(k, carry):
        tb = 2 * k + 1
        ta = 2 * k + 2
        dma_d(tb, dst_pb, sem_db, False)
        dma_s(dst_pa, sem_sa, False)
        dma_s(dst_pb, sem_sb, True)

        @pl.when(ta < NFULL_D)
        def _():
            dma_d(ta, dst_pa, sem_da, True)
            dma_d(ta, dst_pa, sem_da, False)

        dma_s(dst_pb, sem_sb, False)

        @pl.when(ta < NFULL_D)
        def _():
            dma_s(dst_pa, sem_sa, True)

            @pl.when(ta + 1 < NFULL_D)
            def _():
                dma_d(ta + 1, dst_pb, sem_db, True)

        return carry

    lax.fori_loop(0, NFULL_D // 2, pair, 0)
    dma_s(dst_pa, sem_sa, False)

    base = pl.multiple_of(base0 + NFULL_D * CH, 8)
    pltpu.sync_copy(dst_hbm.at[pl.ds(base, TAIL_D)], dst_t)
    pltpu.sync_copy(ones_t, acc.at[dst_t], add=True)
    plsc.subcore_barrier()
    pltpu.sync_copy(acc.at[pl.ds(row0, RQ)], out_hbm.at[pl.ds(c * N + row0, RQ)])

    @pl.when(s == NS - 1)
    def _():
        extra = NS * RQ
        nex = N - extra
        pltpu.sync_copy(acc.at[pl.ds(extra, nex)],
                        out_hbm.at[pl.ds(c * N + extra, nex)])


CHK = 80
NCHS = EPSC // CHK
NTRI = (NCHS - 2) // 3


@functools.cache
def _row_pass_built():
    return pl.kernel(
        _row_body,
        mesh=_get_mesh(),
        out_type=[
            jax.ShapeDtypeStruct((N, DH), jnp.float32),
            jax.ShapeDtypeStruct((N, DH), jnp.float32),
        ],
        scratch_types=[
            pltpu.VMEM((EPSC,), jnp.int32),
            pltpu.VMEM((CHK,), jnp.int32),
            pltpu.VMEM((CHK,), jnp.int32),
            pltpu.VMEM((CHK,), jnp.int32),
            pltpu.VMEM((CHK, DH), jnp.float32),
            pltpu.VMEM((CHK, DH), jnp.float32),
            pltpu.VMEM((CHK, DH), jnp.float32),
            pltpu.VMEM((16, DH), jnp.float32),
            pltpu.VMEM_SHARED((N, DH), jnp.float32),
            pltpu.SemaphoreType.DMA,
            pltpu.SemaphoreType.DMA,
            pltpu.SemaphoreType.DMA,
            pltpu.SemaphoreType.DMA,
            pltpu.SemaphoreType.DMA,
            pltpu.SemaphoreType.DMA,
        ],
    )


def _row_body(hws_a, hws_b, src_hbm, dst_hbm, out_a, out_b,
              src_v, dst_p0, dst_p1, dst_p2, rows_p0, rows_p1, rows_p2,
              zbuf, acc,
              sem_g0, sem_g1, sem_g2, sem_s0, sem_s1, sem_s2):
    c = lax.axis_index("c")
    s = lax.axis_index("s")

    def zfill(r, carry):
        for j in range(DH // 16):
            zbuf[r, pl.ds(j * 16, 16)] = jnp.zeros((16,), jnp.float32)
        return carry

    lax.fori_loop(0, 16, zfill, 0)
    row0 = pl.multiple_of(s * RQ, 8)
    nsteps = jnp.where(s == NS - 1, (N - (NS - 1) * RQ) // 16, RQ // 16)

    def zstep(t, carry):
        pltpu.sync_copy(zbuf, acc.at[pl.ds(row0 + t * 16, 16)])
        return carry

    lax.fori_loop(0, nsteps, zstep, 0)

    base_e = pl.multiple_of(s * EPSC, 8)
    pltpu.sync_copy(src_hbm.at[pl.ds(base_e, EPSC)], src_v)
    plsc.subcore_barrier()

    def run(hws_hbm, out_hbm):
        def dma_g(t, dstb, rows, semg, issue):
            off = pl.multiple_of(base_e + t * CHK, 8)
            idx = src_v.at[pl.ds(pl.multiple_of(t * CHK, 8), CHK)]
            if issue:
                pltpu.async_copy(dst_hbm.at[pl.ds(off, CHK)], dstb, semg)
                pltpu.async_copy(hws_hbm.at[idx], rows, semg)
            else:
                pltpu.make_async_copy(dst_hbm.at[pl.ds(off, CHK)], dstb, semg).wait()
                pltpu.make_async_copy(hws_hbm.at[idx], rows, semg).wait()

        def dma_s(dstb, rows, sems, issue):
            if issue:
                pltpu.async_copy(rows, acc.at[dstb], sems, add=True)
            else:
                pltpu.make_async_copy(rows, acc.at[dstb], sems).wait()

        bufs = ((dst_p0, rows_p0, sem_g0, sem_s0),
                (dst_p1, rows_p1, sem_g1, sem_s1),
                (dst_p2, rows_p2, sem_g2, sem_s2))

        def step(t, b, bprev):
            dma_g(t, b[0], b[1], b[2], False)
            dma_s(bprev[0], bprev[1], bprev[3], False)
            dma_s(b[0], b[1], b[3], True)

            @pl.when(t + 2 < NCHS)
            def _():
                dma_g(t + 2, bprev[0], bprev[1], bprev[2], True)

        dma_g(0, dst_p0, rows_p0, sem_g0, True)
        dma_g(1, dst_p1, rows_p1, sem_g1, True)
        dma_g(0, dst_p0, rows_p0, sem_g0, False)
        dma_s(dst_p0, rows_p0, sem_s0, True)
        dma_g(2, dst_p2, rows_p2, sem_g2, True)
        step(1, bufs[1], bufs[0])

        def tri(k, carry):
            t = 3 * k + 2
            step(t, bufs[2], bufs[1])
            step(t + 1, bufs[0], bufs[2])
            step(t + 2, bufs[1], bufs[0])
            return carry

        lax.fori_loop(0, NTRI, tri, 0)
        dma_s(dst_p1, rows_p1, sem_s1, False)
        plsc.subcore_barrier()
        pltpu.sync_copy(acc.at[pl.ds(row0, RQ)], out_hbm.at[pl.ds(row0, RQ)])

        @pl.when(s == NS - 1)
        def _():
            extra = NS * RQ
            nex = N - extra
            pltpu.sync_copy(acc.at[pl.ds(extra, nex)],
                            out_hbm.at[pl.ds(extra, nex)])

    @pl.when(c == 0)
    def _():
        run(hws_a, out_a)

    @pl.when(c == 1)
    def _():
        run(hws_b, out_b)


BN = 1000
GRID = N // BN
_CONTRACT = (((1,), (1,)), ((), ()))


def _mm(x, w):
    return lax.dot_general(x, w, _CONTRACT, preferred_element_type=jnp.float32)


def _tc_h0_body(x_ref, win_ref, bin_ref, h0_ref):
    h0_ref[...] = jnp.maximum(_mm(x_ref[...], win_ref[...]) + bin_ref[...], 0.0)


_tc_h0 = pl.pallas_call(
    _tc_h0_body,
    grid=(GRID,),
    in_specs=[
        pl.BlockSpec((BN, D), lambda i: (i, 0)),
        pl.BlockSpec((D, D), lambda i: (0, 0)),
        pl.BlockSpec((1, D), lambda i: (0, 0)),
    ],
    out_specs=pl.BlockSpec((BN, D), lambda i: (i, 0)),
    out_shape=jax.ShapeDtypeStruct((N, D), jnp.float32),
)


def _tc_scale_body(h0_ref, w1_ref, degp_ref, dinv_ref, hwsa_ref, hwsb_ref):
    dp = degp_ref[...]
    deg = dp[0][:, :1] + dp[1][:, :1] + 1.0
    dinv = lax.rsqrt(deg)
    hw = _mm(h0_ref[...], w1_ref[...])
    hws = hw * dinv
    dinv_ref[...] = jnp.broadcast_to(dinv, (BN, DH))
    hwsa_ref[...] = hws[:, :DH]
    hwsb_ref[...] = hws[:, DH:]


_tc_scale = pl.pallas_call(
    _tc_scale_body,
    grid=(GRID,),
    in_specs=[
        pl.BlockSpec((BN, D), lambda i: (i, 0)),
        pl.BlockSpec((D, D), lambda i: (0, 0)),
        pl.BlockSpec((2, BN, DEGW), lambda i: (0, i, 0)),
    ],
    out_specs=[
        pl.BlockSpec((BN, DH), lambda i: (i, 0)),
        pl.BlockSpec((BN, DH), lambda i: (i, 0)),
        pl.BlockSpec((BN, DH), lambda i: (i, 0)),
    ],
    out_shape=[
        jax.ShapeDtypeStruct((N, DH), jnp.float32),
        jax.ShapeDtypeStruct((N, DH), jnp.float32),
        jax.ShapeDtypeStruct((N, DH), jnp.float32),
    ],
)


def _layer_tail(sa, sb, hwsa, hwsb, dinv, hprev, bc, g, bl):
    conv = jnp.concatenate([sa + hwsa, sb + hwsb], axis=1) * dinv + bc
    t = hprev + jnp.maximum(conv, 0.0)
    mu = jnp.mean(t, axis=1, keepdims=True)
    var = jnp.mean((t - mu) ** 2, axis=1, keepdims=True)
    return (t - mu) * lax.rsqrt(var + EPS) * g + bl


def _tc_mid_body(sa_ref, sb_ref, hwsa_ref, hwsb_ref, dinv_ref, hprev_ref,
                 bc_ref, g_ref, bl_ref, w2_ref,
                 h1_ref, h2a_ref, h2b_ref):
    dinv = dinv_ref[...][:, :1]
    h1 = _layer_tail(sa_ref[...], sb_ref[...], hwsa_ref[...], hwsb_ref[...],
                     dinv, hprev_ref[...], bc_ref[...], g_ref[...], bl_ref[...])
    h1_ref[...] = h1
    hw2 = _mm(h1, w2_ref[...])
    hws2 = hw2 * dinv
    h2a_ref[...] = hws2[:, :DH]
    h2b_ref[...] = hws2[:, DH:]


_tc_mid = pl.pallas_call(
    _tc_mid_body,
    grid=(GRID,),
    in_specs=[
        pl.BlockSpec((BN, DH), lambda i: (i, 0)),
        pl.BlockSpec((BN, DH), lambda i: (i, 0)),
        pl.BlockSpec((BN, DH), lambda i: (i, 0)),
        pl.BlockSpec((BN, DH), lambda i: (i, 0)),
        pl.BlockSpec((BN, DH), lambda i: (i, 0)),
        pl.BlockSpec((BN, D), lambda i: (i, 0)),
        pl.BlockSpec((1, D), lambda i: (0, 0)),
        pl.BlockSpec((1, D), lambda i: (0, 0)),
        pl.BlockSpec((1, D), lambda i: (0, 0)),
        pl.BlockSpec((D, D), lambda i: (0, 0)),
    ],
    out_specs=[
        pl.BlockSpec((BN, D), lambda i: (i, 0)),
        pl.BlockSpec((BN, DH), lambda i: (i, 0)),
        pl.BlockSpec((BN, DH), lambda i: (i, 0)),
    ],
    out_shape=[
        jax.ShapeDtypeStruct((N, D), jnp.float32),
        jax.ShapeDtypeStruct((N, DH), jnp.float32),
        jax.ShapeDtypeStruct((N, DH), jnp.float32),
    ],
)


def _tc_fin_body(sa_ref, sb_ref, hwsa_ref, hwsb_ref, dinv_ref, hprev_ref,
                 bc_ref, g_ref, bl_ref, wo_ref, bo_ref, out_ref):
    dinv = dinv_ref[...][:, :1]
    h2 = _layer_tail(sa_ref[...], sb_ref[...], hwsa_ref[...], hwsb_ref[...],
                     dinv, hprev_ref[...], bc_ref[...], g_ref[...], bl_ref[...])
    out_ref[...] = _mm(h2, wo_ref[...]) + bo_ref[...]


_tc_fin = pl.pallas_call(
    _tc_fin_body,
    grid=(GRID,),
    in_specs=[
        pl.BlockSpec((BN, DH), lambda i: (i, 0)),
        pl.BlockSpec((BN, DH), lambda i: (i, 0)),
        pl.BlockSpec((BN, DH), lambda i: (i, 0)),
        pl.BlockSpec((BN, DH), lambda i: (i, 0)),
        pl.BlockSpec((BN, DH), lambda i: (i, 0)),
        pl.BlockSpec((BN, D), lambda i: (i, 0)),
        pl.BlockSpec((1, D), lambda i: (0, 0)),
        pl.BlockSpec((1, D), lambda i: (0, 0)),
        pl.BlockSpec((1, D), lambda i: (0, 0)),
        pl.BlockSpec((D, D), lambda i: (0, 0)),
        pl.BlockSpec((1, D), lambda i: (0, 0)),
    ],
    out_specs=pl.BlockSpec((BN, D), lambda i: (i, 0)),
    out_shape=jax.ShapeDtypeStruct((N, D), jnp.float32),
)


def kernel(x, edge_index, W_in, b_in, W_c1, b_c1, g_ln1, b_ln1,
           W_c2, b_c2, g_ln2, b_ln2, W_out, b_out):
    src = edge_index[0]
    dst = edge_index[1]

    degp = _deg_pass_built()(dst).reshape(2, N, DEGW)
    h0 = _tc_h0(x, W_in, b_in.reshape(1, D))
    dinv, hws1a, hws1b = _tc_scale(h0, W_c1, degp)
    s1a, s1b = _row_pass_built()(hws1a, hws1b, src, dst)
    h1, hws2a, hws2b = _tc_mid(
        s1a, s1b, hws1a, hws1b, dinv, h0,
        b_c1.reshape(1, D), g_ln1.reshape(1, D), b_ln1.reshape(1, D), W_c2)
    s2a, s2b = _row_pass_built()(hws2a, hws2b, src, dst)
    out = _tc_fin(
        s2a, s2b, hws2a, hws2b, dinv, h1,
        b_c2.reshape(1, D), g_ln2.reshape(1, D), b_ln2.reshape(1, D),
        W_out, b_out.reshape(1, D))
    return out


# speedup vs baseline: 18.8252x; 1.0005x over previous
import functools

import jax
import jax.numpy as jnp
from jax import lax
from jax.experimental import pallas as pl
from jax.experimental.pallas import tpu as pltpu
from jax.experimental.pallas import tpu_sc as plsc

N = 10000
E = 160000
D = 256
DH = 128
EPS = 1e-5

NC = 2
NS = 16
RQ = 624
EPSC = E // NS
CH = 128
EPW = E // (NC * NS)
NFULL_D = EPW // CH
TAIL_D = EPW - NFULL_D * CH
DEGW = 128

@functools.cache
def _get_mesh():
    return plsc.VectorSubcoreMesh(
        core_axis_name="c", subcore_axis_name="s", num_cores=NC, num_subcores=NS
    )


@functools.cache
def _deg_pass_built():
    return pl.kernel(
        _deg_body,
        mesh=_get_mesh(),
        out_type=jax.ShapeDtypeStruct((2 * N, DEGW), jnp.float32),
        scratch_types=[
            pltpu.VMEM((CH,), jnp.int32),
            pltpu.VMEM((CH,), jnp.int32),
            pltpu.VMEM((TAIL_D,), jnp.int32),
            pltpu.VMEM((CH, DEGW), jnp.float32),
            pltpu.VMEM((TAIL_D, DEGW), jnp.float32),
            pltpu.VMEM((16, DEGW), jnp.float32),
            pltpu.VMEM_SHARED((N, DEGW), jnp.float32),
            pltpu.SemaphoreType.DMA,
            pltpu.SemaphoreType.DMA,
            pltpu.SemaphoreType.DMA,
            pltpu.SemaphoreType.DMA,
        ],
    )


def _deg_body(dst_hbm, out_hbm, dst_pa, dst_pb, dst_t, ones_v, ones_t, zbuf, acc,
              sem_da, sem_db, sem_sa, sem_sb):
    c = lax.axis_index("c")
    s = lax.axis_index("s")

    def zfill(r, carry):
        for j in range(DEGW // 16):
            zbuf[r, pl.ds(j * 16, 16)] = jnp.zeros((16,), jnp.float32)
        return carry

    lax.fori_loop(0, 16, zfill, 0)

    def ofill(r, carry):
        for j in range(DEGW // 16):
            ones_v[r, pl.ds(j * 16, 16)] = jnp.full((16,), 1.0, jnp.float32)
        return carry

    lax.fori_loop(0, CH, ofill, 0)
    for r in range(TAIL_D):
        for j in range(DEGW // 16):
            ones_t[r, pl.ds(j * 16, 16)] = jnp.full((16,), 1.0, jnp.float32)

    row0 = pl.multiple_of(s * RQ, 8)
    nsteps = jnp.where(s == NS - 1, (N - (NS - 1) * RQ) // 16, RQ // 16)

    def zstep(t, carry):
        pltpu.sync_copy(zbuf, acc.at[pl.ds(row0 + t * 16, 16)])
        return carry

    lax.fori_loop(0, nsteps, zstep, 0)
    plsc.subcore_barrier()

    base0 = (s * NC + c) * EPW

    def dma_d(t, dstb, semd, issue):
        base = pl.multiple_of(base0 + t * CH, 8)
        if issue:
            pltpu.async_copy(dst_hbm.at[pl.ds(base, CH)], dstb, semd)
        else:
            pltpu.make_async_copy(dst_hbm.at[pl.ds(base, CH)], dstb, semd).wait()

    def dma_s(dstb, sems, issue):
        if issue:
            pltpu.async_copy(ones_v, acc.at[dstb], sems, add=True)
        else:
            pltpu.make_async_copy(ones_v, acc.at[dstb], sems).wait()

    dma_d(0, dst_pa, sem_da, True)
    dma_d(0, dst_pa, sem_da, False)
    dma_s(dst_pa, sem_sa, True)
    dma_d(1, dst_pb, sem_db, True)

    def pair(k, carry):
        tb = 2 * k + 1
        ta = 2 * k + 2
        dma_d(tb, dst_pb, sem_db, False)
        dma_s(dst_pa, sem_sa, False)
        dma_s(dst_pb, sem_sb, True)

        @pl.when(ta < NFULL_D)
        def _():
            dma_d(ta, dst_pa, sem_da, True)
            dma_d(ta, dst_pa, sem_da, False)

        dma_s(dst_pb, sem_sb, False)

        @pl.when(ta < NFULL_D)
        def _():
            dma_s(dst_pa, sem_sa, True)

            @pl.when(ta + 1 < NFULL_D)
            def _():
                dma_d(ta + 1, dst_pb, sem_db, True)

        return carry

    lax.fori_loop(0, NFULL_D // 2, pair, 0)
    dma_s(dst_pa, sem_sa, False)

    base = pl.multiple_of(base0 + NFULL_D * CH, 8)
    pltpu.sync_copy(dst_hbm.at[pl.ds(base, TAIL_D)], dst_t)
    pltpu.sync_copy(ones_t, acc.at[dst_t], add=True)
    plsc.subcore_barrier()
    pltpu.sync_copy(acc.at[pl.ds(row0, RQ)], out_hbm.at[pl.ds(c * N + row0, RQ)])

    @pl.when(s == NS - 1)
    def _():
        extra = NS * RQ
        nex = N - extra
        pltpu.sync_copy(acc.at[pl.ds(extra, nex)],
                        out_hbm.at[pl.ds(c * N + extra, nex)])


CHK = 80
NCHS = EPSC // CHK
NTRI = (NCHS - 2) // 3


@functools.cache
def _row_pass_built():
    return pl.kernel(
        _row_body,
        mesh=_get_mesh(),
        out_type=[
            jax.ShapeDtypeStruct((N, DH), jnp.float32),
            jax.ShapeDtypeStruct((N, DH), jnp.float32),
        ],
        scratch_types=[
            pltpu.VMEM((EPSC,), jnp.int32),
            pltpu.VMEM((CHK,), jnp.int32),
            pltpu.VMEM((CHK,), jnp.int32),
            pltpu.VMEM((CHK,), jnp.int32),
            pltpu.VMEM((CHK, DH), jnp.float32),
            pltpu.VMEM((CHK, DH), jnp.float32),
            pltpu.VMEM((CHK, DH), jnp.float32),
            pltpu.VMEM((16, DH), jnp.float32),
            pltpu.VMEM_SHARED((N, DH), jnp.float32),
            pltpu.SemaphoreType.DMA,
            pltpu.SemaphoreType.DMA,
            pltpu.SemaphoreType.DMA,
            pltpu.SemaphoreType.DMA,
            pltpu.SemaphoreType.DMA,
            pltpu.SemaphoreType.DMA,
        ],
    )


def _row_body(hws_a, hws_b, src_hbm, dst_hbm, out_a, out_b,
              src_v, dst_p0, dst_p1, dst_p2, rows_p0, rows_p1, rows_p2,
              zbuf, acc,
              sem_g0, sem_g1, sem_g2, sem_s0, sem_s1, sem_s2):
    c = lax.axis_index("c")
    s = lax.axis_index("s")

    def zfill(r, carry):
        for j in range(DH // 16):
            zbuf[r, pl.ds(j * 16, 16)] = jnp.zeros((16,), jnp.float32)
        return carry

    lax.fori_loop(0, 16, zfill, 0)
    row0 = pl.multiple_of(s * RQ, 8)
    nsteps = jnp.where(s == NS - 1, (N - (NS - 1) * RQ) // 16, RQ // 16)

    def zstep(t, carry):
        pltpu.sync_copy(zbuf, acc.at[pl.ds(row0 + t * 16, 16)])
        return carry

    lax.fori_loop(0, nsteps, zstep, 0)

    base_e = pl.multiple_of(s * EPSC, 8)
    pltpu.sync_copy(src_hbm.at[pl.ds(base_e, EPSC)], src_v)
    plsc.subcore_barrier()

    def run(hws_hbm, out_hbm):
        def dma_g(t, dstb, rows, semg, issue):
            off = pl.multiple_of(base_e + t * CHK, 8)
            idx = src_v.at[pl.ds(pl.multiple_of(t * CHK, 8), CHK)]
            if issue:
                pltpu.async_copy(dst_hbm.at[pl.ds(off, CHK)], dstb, semg)
                pltpu.async_copy(hws_hbm.at[idx], rows, semg)
            else:
                pltpu.make_async_copy(dst_hbm.at[pl.ds(off, CHK)], dstb, semg).wait()
                pltpu.make_async_copy(hws_hbm.at[idx], rows, semg).wait()

        def dma_s(dstb, rows, sems, issue):
            if issue:
                pltpu.async_copy(rows, acc.at[dstb], sems, add=True)
            else:
                pltpu.make_async_copy(rows, acc.at[dstb], sems).wait()

        bufs = ((dst_p0, rows_p0, sem_g0, sem_s0),
                (dst_p1, rows_p1, sem_g1, sem_s1),
                (dst_p2, rows_p2, sem_g2, sem_s2))

        def step(t, b, bprev):
            dma_g(t, b[0], b[1], b[2], False)
            dma_s(bprev[0], bprev[1], bprev[3], False)
            dma_s(b[0], b[1], b[3], True)

            @pl.when(t + 2 < NCHS)
            def _():
                dma_g(t + 2, bprev[0], bprev[1], bprev[2], True)

        dma_g(0, dst_p0, rows_p0, sem_g0, True)
        dma_g(1, dst_p1, rows_p1, sem_g1, True)
        dma_g(0, dst_p0, rows_p0, sem_g0, False)
        dma_s(dst_p0, rows_p0, sem_s0, True)
        dma_g(2, dst_p2, rows_p2, sem_g2, True)
        step(1, bufs[1], bufs[0])

        def tri(k, carry):
            t = 3 * k + 2
            step(t, bufs[2], bufs[1])
            step(t + 1, bufs[0], bufs[2])
            step(t + 2, bufs[1], bufs[0])
            return carry

        lax.fori_loop(0, NTRI, tri, 0)
        dma_s(dst_p1, rows_p1, sem_s1, False)
        plsc.subcore_barrier()
        pltpu.sync_copy(acc.at[pl.ds(row0, RQ)], out_hbm.at[pl.ds(row0, RQ)])

        @pl.when(s == NS - 1)
        def _():
            extra = NS * RQ
            nex = N - extra
            pltpu.sync_copy(acc.at[pl.ds(extra, nex)],
                            out_hbm.at[pl.ds(extra, nex)])

    @pl.when(c == 0)
    def _():
        run(hws_a, out_a)

    @pl.when(c == 1)
    def _():
        run(hws_b, out_b)


BN = 1000
GRID = N // BN
_CONTRACT = (((1,), (1,)), ((), ()))


def _mm(x, w):
    return lax.dot_general(x, w, _CONTRACT, preferred_element_type=jnp.float32)


def _tc_h0_body(x_ref, win_ref, bin_ref, h0_ref):
    h0_ref[...] = jnp.maximum(_mm(x_ref[...], win_ref[...]) + bin_ref[...], 0.0)


_tc_h0 = pl.pallas_call(
    _tc_h0_body,
    grid=(GRID,),
    in_specs=[
        pl.BlockSpec((BN, D), lambda i: (i, 0)),
        pl.BlockSpec((D, D), lambda i: (0, 0)),
        pl.BlockSpec((1, D), lambda i: (0, 0)),
    ],
    out_specs=pl.BlockSpec((BN, D), lambda i: (i, 0)),
    out_shape=jax.ShapeDtypeStruct((N, D), jnp.float32),
)


def _tc_scale_body(h0_ref, w1_ref, degp_ref, dinv_ref, hwsa_ref, hwsb_ref):
    dp = degp_ref[...]
    deg = dp[0][:, :1] + dp[1][:, :1] + 1.0
    dinv = lax.rsqrt(deg)
    hw = _mm(h0_ref[...], w1_ref[...])
    hws = hw * dinv
    dinv_ref[...] = jnp.broadcast_to(dinv, (BN, DH))
    hwsa_ref[...] = hws[:, :DH]
    hwsb_ref[...] = hws[:, DH:]


_tc_scale = pl.pallas_call(
    _tc_scale_body,
    grid=(GRID,),
    in_specs=[
        pl.BlockSpec((BN, D), lambda i: (i, 0)),
        pl.BlockSpec((D, D), lambda i: (0, 0)),
        pl.BlockSpec((2, BN, DEGW), lambda i: (0, i, 0)),
    ],
    out_specs=[
        pl.BlockSpec((BN, DH), lambda i: (i, 0)),
        pl.BlockSpec((BN, DH), lambda i: (i, 0)),
        pl.BlockSpec((BN, DH), lambda i: (i, 0)),
    ],
    out_shape=[
        jax.ShapeDtypeStruct((N, DH), jnp.float32),
        jax.ShapeDtypeStruct((N, DH), jnp.float32),
        jax.ShapeDtypeStruct((N, DH), jnp.float32),
    ],
)


def _layer_tail(sa, sb, hwsa, hwsb, dinv, hprev, bc, g, bl):
    conv = jnp.concatenate([sa + hwsa, sb + hwsb], axis=1) * dinv + bc
    t = hprev + jnp.maximum(conv, 0.0)
    mu = jnp.mean(t, axis=1, keepdims=True)
    var = jnp.mean((t - mu) ** 2, axis=1, keepdims=True)
    return (t - mu) * lax.rsqrt(var + EPS) * g + bl


def _tc_mid_body(sa_ref, sb_ref, hwsa_ref, hwsb_ref, dinv_ref, hprev_ref,
                 bc_ref, g_ref, bl_ref, w2_ref,
                 h1_ref, h2a_ref, h2b_ref):
    dinv = dinv_ref[...][:, :1]
    h1 = _layer_tail(sa_ref[...], sb_ref[...], hwsa_ref[...], hwsb_ref[...],
                     dinv, hprev_ref[...], bc_ref[...], g_ref[...], bl_ref[...])
    h1_ref[...] = h1
    hw2 = _mm(h1, w2_ref[...])
    hws2 = hw2 * dinv
    h2a_ref[...] = hws2[:, :DH]
    h2b_ref[...] = hws2[:, DH:]


_tc_mid = pl.pallas_call(
    _tc_mid_body,
    grid=(GRID,),
    in_specs=[
        pl.BlockSpec((BN, DH), lambda i: (i, 0)),
        pl.BlockSpec((BN, DH), lambda i: (i, 0)),
        pl.BlockSpec((BN, DH), lambda i: (i, 0)),
        pl.BlockSpec((BN, DH), lambda i: (i, 0)),
        pl.BlockSpec((BN, DH), lambda i: (i, 0)),
        pl.BlockSpec((BN, D), lambda i: (i, 0)),
        pl.BlockSpec((1, D), lambda i: (0, 0)),
        pl.BlockSpec((1, D), lambda i: (0, 0)),
        pl.BlockSpec((1, D), lambda i: (0, 0)),
        pl.BlockSpec((D, D), lambda i: (0, 0)),
    ],
    out_specs=[
        pl.BlockSpec((BN, D), lambda i: (i, 0)),
        pl.BlockSpec((BN, DH), lambda i: (i, 0)),
        pl.BlockSpec((BN, DH), lambda i: (i, 0)),
    ],
    out_shape=[
        jax.ShapeDtypeStruct((N, D), jnp.float32),
        jax.ShapeDtypeStruct((N, DH), jnp.float32),
        jax.ShapeDtypeStruct((N, DH), jnp.float32),
    ],
)


def _tc_fin_body(sa_ref, sb_ref, hwsa_ref, hwsb_ref, dinv_ref, hprev_ref,
                 bc_ref, g_ref, bl_ref, wo_ref, bo_ref, out_ref):
    dinv = dinv_ref[...][:, :1]
    h2 = _layer_tail(sa_ref[...], sb_ref[...], hwsa_ref[...], hwsb_ref[...],
                     dinv, hprev_ref[...], bc_ref[...], g_ref[...], bl_ref[...])
    out_ref[...] = _mm(h2, wo_ref[...]) + bo_ref[...]


_tc_fin = pl.pallas_call(
    _tc_fin_body,
    grid=(GRID,),
    in_specs=[
        pl.BlockSpec((BN, DH), lambda i: (i, 0)),
        pl.BlockSpec((BN, DH), lambda i: (i, 0)),
        pl.BlockSpec((BN, DH), lambda i: (i, 0)),
        pl.BlockSpec((BN, DH), lambda i: (i, 0)),
        pl.BlockSpec((BN, DH), lambda i: (i, 0)),
        pl.BlockSpec((BN, D), lambda i: (i, 0)),
        pl.BlockSpec((1, D), lambda i: (0, 0)),
        pl.BlockSpec((1, D), lambda i: (0, 0)),
        pl.BlockSpec((1, D), lambda i: (0, 0)),
        pl.BlockSpec((D, D), lambda i: (0, 0)),
        pl.BlockSpec((1, D), lambda i: (0, 0)),
    ],
    out_specs=pl.BlockSpec((BN, D), lambda i: (i, 0)),
    out_shape=jax.ShapeDtypeStruct((N, D), jnp.float32),
)


def kernel(x, edge_index, W_in, b_in, W_c1, b_c1, g_ln1, b_ln1,
           W_c2, b_c2, g_ln2, b_ln2, W_out, b_out):
    src = edge_index[0]
    dst = edge_index[1]

    degp = _deg_pass_built()(dst).reshape(2, N, DEGW)
    h0 = _tc_h0(x, W_in, b_in.reshape(1, D))
    dinv, hws1a, hws1b = _tc_scale(h0, W_c1, degp)
    s1a, s1b = _row_pass_built()(hws1a, hws1b, src, dst)
    h1, hws2a, hws2b = _tc_mid(
        s1a, s1b, hws1a, hws1b, dinv, h0,
        b_c1.reshape(1, D), g_ln1.reshape(1, D), b_ln1.reshape(1, D), W_c2)
    s2a, s2b = _row_pass_built()(hws2a, hws2b, src, dst)
    out = _tc_fin(
        s2a, s2b, hws2a, hws2b, dinv, h1,
        b_c2.reshape(1, D), g_ln2.reshape(1, D), b_ln2.reshape(1, D),
        W_out, b_out.reshape(1, D))
    return out
